# speedup vs baseline: 1.0049x; 1.0049x over previous
import jax
import jax.numpy as jnp
from jax.experimental import pallas as pl

N = 10000
E = 320000
D = 128
DE = 16
C_NORM = 32.0

EDGE_BLK = 2560
NODE_BLK = 2000


def _edge_body(t1_ref, w2_ref, b2_ref, wc1_ref, bc1_ref, wc2_ref, bc2_ref,
               m2_ref, cw_ref):
    t1 = t1_ref[...]
    m = jax.nn.silu(t1)
    m2 = jax.nn.silu(
        jnp.dot(m, w2_ref[...], preferred_element_type=jnp.float32)
        + b2_ref[...])
    c1 = jax.nn.silu(
        jnp.dot(m2, wc1_ref[...], preferred_element_type=jnp.float32)
        + bc1_ref[...])
    cw = jnp.dot(c1, wc2_ref[...], preferred_element_type=jnp.float32) \
        + bc2_ref[...]
    m2_ref[...] = m2
    cw_ref[...] = cw


def _node_body(h_ref, agg_ref, wn1a_ref, wn1b_ref, bn1_ref, wn2_ref, bn2_ref,
               out_ref):
    h = h_ref[...]
    t = (jnp.dot(h, wn1a_ref[...], preferred_element_type=jnp.float32)
         + jnp.dot(agg_ref[...], wn1b_ref[...],
                   preferred_element_type=jnp.float32)
         + bn1_ref[...])
    out_ref[...] = h + jnp.dot(jax.nn.silu(t), wn2_ref[...],
                               preferred_element_type=jnp.float32) \
        + bn2_ref[...]


def kernel(atom_feats, coord, edge_index, edge_type_ids, atom_table,
           edge_table, W1, b1, W2, b2, Wc1, bc1, Wc2, bc2, Wn1, bn1, Wn2,
           bn2):
    row = edge_index[0]
    col = edge_index[1]

    h = jnp.take(atom_table, atom_feats, axis=0)
    hp_r = h @ W1[:D]
    hp_c = h @ W1[D:2 * D]
    w1d = W1[2 * D]
    eap = edge_table @ W1[2 * D + 1:] + b1

    edge_attr = jnp.take(edge_table, edge_type_ids, axis=0)

    rel = coord[row] - coord[col]
    d2 = jnp.sum(rel * rel, axis=-1, keepdims=True)
    t1 = (jnp.take(hp_r, row, axis=0) + jnp.take(hp_c, col, axis=0)
          + d2 * w1d + jnp.take(eap, edge_type_ids, axis=0))

    m2, cw = pl.pallas_call(
        _edge_body,
        grid=(E // EDGE_BLK,),
        in_specs=[
            pl.BlockSpec((EDGE_BLK, D), lambda i: (i, 0)),
            pl.BlockSpec((D, D), lambda i: (0, 0)),
            pl.BlockSpec((D,), lambda i: (0,)),
            pl.BlockSpec((D, D), lambda i: (0, 0)),
            pl.BlockSpec((D,), lambda i: (0,)),
            pl.BlockSpec((D, 1), lambda i: (0, 0)),
            pl.BlockSpec((1,), lambda i: (0,)),
        ],
        out_specs=[
            pl.BlockSpec((EDGE_BLK, D), lambda i: (i, 0)),
            pl.BlockSpec((EDGE_BLK, 1), lambda i: (i, 0)),
        ],
        out_shape=[
            jax.ShapeDtypeStruct((E, D), jnp.float32),
            jax.ShapeDtypeStruct((E, 1), jnp.float32),
        ],
    )(t1, W2, b2, Wc1, bc1, Wc2, bc2)

    trans = rel * cw
    coord_agg = jax.ops.segment_sum(trans, row, num_segments=N)
    coord_out = coord + coord_agg / C_NORM

    agg = jax.ops.segment_sum(m2, row, num_segments=N)

    h_out = pl.pallas_call(
        _node_body,
        grid=(N // NODE_BLK,),
        in_specs=[
            pl.BlockSpec((NODE_BLK, D), lambda i: (i, 0)),
            pl.BlockSpec((NODE_BLK, D), lambda i: (i, 0)),
            pl.BlockSpec((D, D), lambda i: (0, 0)),
            pl.BlockSpec((D, D), lambda i: (0, 0)),
            pl.BlockSpec((D,), lambda i: (0,)),
            pl.BlockSpec((D, D), lambda i: (0, 0)),
            pl.BlockSpec((D,), lambda i: (0,)),
        ],
        out_specs=pl.BlockSpec((NODE_BLK, D), lambda i: (i, 0)),
        out_shape=jax.ShapeDtypeStruct((N, D), jnp.float32),
    )(h, agg, Wn1[:D], Wn1[D:], bn1, Wn2, bn2)

    return (h_out, coord_out, edge_attr)


# speedup vs baseline: 3.0314x; 3.0165x over previous
import jax
import jax.numpy as jnp
from jax import lax
from jax.experimental import pallas as pl
from jax.experimental.pallas import tpu as pltpu
from jax.experimental.pallas import tpu_sc as plsc

N = 10000
E = 320000
D = 128
DE = 16
C_NORM = 32.0
NTYPES = 16

EDGE_BLK = 2560
NODE_BLK = 2000
TW = 2 * D
CPAD = 16

SC_CORES = 2
SC_TILES = 16
CHUNK = 80
EPC = E // SC_CORES
EPT = EPC // SC_TILES
NCH = EPT // CHUNK
NP = 10240
NPT = NP // SC_TILES


def _gather_body(row_hbm, col_hbm, tr_hbm, tc_hbm, tpre_hbm, relp_hbm,
                 ridx_v, cidx_v, a_v, b_v, t_v, r_v, sem):
    c = lax.axis_index("c")
    s = lax.axis_index("s")
    base0 = c * EPC + s * EPT

    def loop(j, carry):
        base = base0 + j * CHUNK
        pltpu.sync_copy(row_hbm.at[pl.ds(base, CHUNK)], ridx_v)
        pltpu.sync_copy(col_hbm.at[pl.ds(base, CHUNK)], cidx_v)
        cp1 = pltpu.async_copy(tr_hbm.at[ridx_v], a_v, sem)
        cp2 = pltpu.async_copy(tc_hbm.at[cidx_v], b_v, sem)
        cp1.wait()
        cp2.wait()

        def rowop(i, carry2):
            for l in range(D // 16):
                sl = pl.ds(l * 16, 16)
                t_v[i, sl] = a_v[i, sl] + b_v[i, sl]
            sl = pl.ds(D, 16)
            r_v[i, :] = a_v[i, sl] - b_v[i, sl]
            return carry2

        lax.fori_loop(0, CHUNK, rowop, 0)
        pltpu.sync_copy(t_v, tpre_hbm.at[pl.ds(base, CHUNK), :])
        pltpu.sync_copy(r_v, relp_hbm.at[pl.ds(base, CHUNK), :])
        return carry

    lax.fori_loop(0, NCH, loop, 0)


def _sc_gather(row, col, t_r, t_c):
    return pl.kernel(
        _gather_body,
        out_type=[
            jax.ShapeDtypeStruct((E, D), jnp.float32),
            jax.ShapeDtypeStruct((E, CPAD), jnp.float32),
        ],
        mesh=plsc.VectorSubcoreMesh(core_axis_name="c", subcore_axis_name="s",
                                    num_cores=SC_CORES,
                                    num_subcores=SC_TILES),
        scratch_types=[
            pltpu.VMEM((CHUNK,), jnp.int32),
            pltpu.VMEM((CHUNK,), jnp.int32),
            pltpu.VMEM((CHUNK, TW), jnp.float32),
            pltpu.VMEM((CHUNK, TW), jnp.float32),
            pltpu.VMEM((CHUNK, D), jnp.float32),
            pltpu.VMEM((CHUNK, CPAD), jnp.float32),
            pltpu.SemaphoreType.DMA,
        ],
    )(row, col, t_r, t_c)


def _scatter_body(row_hbm, m2_hbm, tr16_hbm, zm_hbm, outm_hbm, outc_hbm,
                  accm, ridx_v, m_v, t16_v, tw_v):
    c = lax.axis_index("c")
    s = lax.axis_index("s")
    base0 = c * EPC + s * EPT

    pltpu.sync_copy(zm_hbm.at[pl.ds(s * NPT, NPT), :],
                    accm.at[pl.ds(s * NPT, NPT), :])
    plsc.subcore_barrier()

    def loop_m(j, carry):
        base = base0 + j * CHUNK
        pltpu.sync_copy(row_hbm.at[pl.ds(base, CHUNK)], ridx_v)
        pltpu.sync_copy(m2_hbm.at[pl.ds(base, CHUNK), :], m_v)
        pltpu.sync_copy(m_v, accm.at[ridx_v], add=True)
        return carry

    lax.fori_loop(0, NCH, loop_m, 0)
    plsc.subcore_barrier()
    pltpu.sync_copy(accm.at[pl.ds(s * NPT, NPT), :],
                    outm_hbm.at[pl.ds(c * NP + s * NPT, NPT), :])
    plsc.subcore_barrier()

    pltpu.sync_copy(zm_hbm.at[pl.ds(s * NPT, NPT), :],
                    accm.at[pl.ds(s * NPT, NPT), :])
    pltpu.sync_copy(zm_hbm.at[pl.ds(0, CHUNK), :], tw_v)
    plsc.subcore_barrier()

    def loop_c(j, carry):
        base = base0 + j * CHUNK
        pltpu.sync_copy(row_hbm.at[pl.ds(base, CHUNK)], ridx_v)
        pltpu.sync_copy(tr16_hbm.at[pl.ds(base, CHUNK), :], t16_v)

        def rowop(i, carry2):
            tw_v[i, pl.ds(0, 16)] = t16_v[i, :]
            return carry2

        lax.fori_loop(0, CHUNK, rowop, 0)
        pltpu.sync_copy(tw_v, accm.at[ridx_v], add=True)
        return carry

    lax.fori_loop(0, NCH, loop_c, 0)
    plsc.subcore_barrier()
    pltpu.sync_copy(accm.at[pl.ds(s * NPT, NPT), :],
                    outc_hbm.at[pl.ds(c * NP + s * NPT, NPT), :])


def _sc_scatter(row, m2, transp, zm):
    return pl.kernel(
        _scatter_body,
        out_type=[
            jax.ShapeDtypeStruct((SC_CORES * NP, D), jnp.float32),
            jax.ShapeDtypeStruct((SC_CORES * NP, D), jnp.float32),
        ],
        mesh=plsc.VectorSubcoreMesh(core_axis_name="c", subcore_axis_name="s",
                                    num_cores=SC_CORES,
                                    num_subcores=SC_TILES),
        scratch_types=[
            pltpu.VMEM_SHARED((NP, D), jnp.float32),
            pltpu.VMEM((CHUNK,), jnp.int32),
            pltpu.VMEM((CHUNK, D), jnp.float32),
            pltpu.VMEM((CHUNK, CPAD), jnp.float32),
            pltpu.VMEM((CHUNK, D), jnp.float32),
        ],
    )(row, m2, transp, zm)


def _prep_body(af_ref, coordp_ref, atab_ref, w1a_ref, w1b_ref,
               h_ref, tr_ref, tc_ref):
    ids = af_ref[0, 0, :]
    oh = (lax.broadcast_in_dim(ids, (NODE_BLK, NTYPES), (0,))
          == lax.broadcasted_iota(jnp.int32, (NODE_BLK, NTYPES), 1)
          ).astype(jnp.float32)
    h = jnp.dot(oh, atab_ref[...], preferred_element_type=jnp.float32)
    h_ref[...] = h
    z = jnp.zeros((NODE_BLK, TW - D - CPAD), jnp.float32)
    cp = coordp_ref[...]
    tr_ref[...] = jnp.concatenate(
        [jnp.dot(h, w1a_ref[...], preferred_element_type=jnp.float32),
         cp, z], axis=1)
    tc_ref[...] = jnp.concatenate(
        [jnp.dot(h, w1b_ref[...], preferred_element_type=jnp.float32),
         cp, z], axis=1)


def _edge_body(tpre_ref, relp_ref, et_ref, eap_ref, w1d_ref, w2_ref, b2_ref,
               wc1_ref, bc1_ref, wc2_ref, bc2_ref, etab_ref,
               m2_ref, tr_ref, ea_ref):
    relp = relp_ref[...]
    d2 = jnp.sum(relp * relp, axis=1, keepdims=True)
    ids = et_ref[0, 0, :]
    oh = (lax.broadcast_in_dim(ids, (EDGE_BLK, 8), (0,))
          == lax.broadcasted_iota(jnp.int32, (EDGE_BLK, 8), 1)
          ).astype(jnp.float32)
    t1 = (tpre_ref[...] + d2 * w1d_ref[...]
          + jnp.dot(oh, eap_ref[...], preferred_element_type=jnp.float32))
    m = jax.nn.silu(t1)
    m2 = jax.nn.silu(
        jnp.dot(m, w2_ref[...], preferred_element_type=jnp.float32)
        + b2_ref[...])
    c1 = jax.nn.silu(
        jnp.dot(m2, wc1_ref[...], preferred_element_type=jnp.float32)
        + bc1_ref[...])
    cw = jnp.dot(c1, wc2_ref[...], preferred_element_type=jnp.float32) \
        + bc2_ref[...]
    m2_ref[...] = m2
    tr_ref[...] = relp * cw
    ea_ref[...] = jnp.dot(oh, etab_ref[...],
                          preferred_element_type=jnp.float32)


def _node_body(h_ref, agg0_ref, agg1_ref, wn1a_ref, wn1b_ref, bn1_ref,
               wn2_ref, bn2_ref, out_ref):
    h = h_ref[...]
    agg = agg0_ref[...] + agg1_ref[...]
    t = (jnp.dot(h, wn1a_ref[...], preferred_element_type=jnp.float32)
         + jnp.dot(agg, wn1b_ref[...], preferred_element_type=jnp.float32)
         + bn1_ref[...])
    out_ref[...] = h + jnp.dot(jax.nn.silu(t), wn2_ref[...],
                               preferred_element_type=jnp.float32) \
        + bn2_ref[...]


def _full(shape):
    return pl.BlockSpec(shape, lambda i: tuple(0 for _ in shape))


def kernel(atom_feats, coord, edge_index, edge_type_ids, atom_table,
           edge_table, W1, b1, W2, b2, Wc1, bc1, Wc2, bc2, Wn1, bn1, Wn2,
           bn2):
    row = edge_index[0]
    col = edge_index[1]

    coordp = jnp.pad(coord, ((0, 0), (0, CPAD - 3)))
    af3 = atom_feats.reshape(N // NODE_BLK, 1, NODE_BLK)
    et3 = edge_type_ids.reshape(E // EDGE_BLK, 1, EDGE_BLK)
    eap8 = jnp.pad(edge_table @ W1[2 * D + 1:] + b1, ((0, 8 - 4), (0, 0)))
    etab8 = jnp.pad(edge_table, ((0, 8 - 4), (0, 0)))
    w1d = W1[2 * D][None, :]

    h, t_r, t_c = pl.pallas_call(
        _prep_body,
        grid=(N // NODE_BLK,),
        in_specs=[
            pl.BlockSpec((1, 1, NODE_BLK), lambda i: (i, 0, 0)),
            pl.BlockSpec((NODE_BLK, CPAD), lambda i: (i, 0)),
            _full((NTYPES, D)),
            _full((D, D)),
            _full((D, D)),
        ],
        out_specs=[
            pl.BlockSpec((NODE_BLK, D), lambda i: (i, 0)),
            pl.BlockSpec((NODE_BLK, TW), lambda i: (i, 0)),
            pl.BlockSpec((NODE_BLK, TW), lambda i: (i, 0)),
        ],
        out_shape=[
            jax.ShapeDtypeStruct((N, D), jnp.float32),
            jax.ShapeDtypeStruct((N, TW), jnp.float32),
            jax.ShapeDtypeStruct((N, TW), jnp.float32),
        ],
    )(af3, coordp, atom_table, W1[:D], W1[D:2 * D])

    tpre, relp = _sc_gather(row, col, t_r, t_c)

    m2, transp, edge_attr = pl.pallas_call(
        _edge_body,
        grid=(E // EDGE_BLK,),
        in_specs=[
            pl.BlockSpec((EDGE_BLK, D), lambda i: (i, 0)),
            pl.BlockSpec((EDGE_BLK, CPAD), lambda i: (i, 0)),
            pl.BlockSpec((1, 1, EDGE_BLK), lambda i: (i, 0, 0)),
            _full((8, D)),
            _full((1, D)),
            _full((D, D)),
            _full((D,)),
            _full((D, D)),
            _full((D,)),
            _full((D, 1)),
            _full((1,)),
            _full((8, DE)),
        ],
        out_specs=[
            pl.BlockSpec((EDGE_BLK, D), lambda i: (i, 0)),
            pl.BlockSpec((EDGE_BLK, CPAD), lambda i: (i, 0)),
            pl.BlockSpec((EDGE_BLK, DE), lambda i: (i, 0)),
        ],
        out_shape=[
            jax.ShapeDtypeStruct((E, D), jnp.float32),
            jax.ShapeDtypeStruct((E, CPAD), jnp.float32),
            jax.ShapeDtypeStruct((E, DE), jnp.float32),
        ],
    )(tpre, relp, et3, eap8, w1d, W2, b2, Wc1, bc1, Wc2, bc2, etab8)

    zm = jnp.zeros((NP, D), jnp.float32)
    outm, outc = _sc_scatter(row, m2, transp, zm)
    aggm = outm.reshape(SC_CORES, NP, D)
    aggc = outc.reshape(SC_CORES, NP, D)
    coord_out = coord + (aggc[0, :N, :3] + aggc[1, :N, :3]) / C_NORM

    h_out = pl.pallas_call(
        _node_body,
        grid=(N // NODE_BLK,),
        in_specs=[
            pl.BlockSpec((NODE_BLK, D), lambda i: (i, 0)),
            pl.BlockSpec((NODE_BLK, D), lambda i: (i, 0)),
            pl.BlockSpec((NODE_BLK, D), lambda i: (i, 0)),
            _full((D, D)),
            _full((D, D)),
            _full((D,)),
            _full((D, D)),
            _full((D,)),
        ],
        out_specs=pl.BlockSpec((NODE_BLK, D), lambda i: (i, 0)),
        out_shape=jax.ShapeDtypeStruct((N, D), jnp.float32),
    )(h, aggm[0, :N], aggm[1, :N], Wn1[:D], Wn1[D:], bn1, Wn2, bn2)

    return (h_out, coord_out, edge_attr)


# speedup vs baseline: 4.0494x; 1.3358x over previous
import jax
import jax.numpy as jnp
from jax import lax
from jax.experimental import pallas as pl
from jax.experimental.pallas import tpu as pltpu
from jax.experimental.pallas import tpu_sc as plsc

N = 10000
E = 320000
D = 128
DE = 16
C_NORM = 32.0
NTYPES = 16

EDGE_BLK = 2560
NODE_BLK = 2000
TW = 2 * D
CPAD = 16

SC_CORES = 2
SC_TILES = 16
CHUNK = 80
EPC = E // SC_CORES
EPT = EPC // SC_TILES
NCH = EPT // CHUNK
NP = 10240
NPT = NP // SC_TILES
GCHUNK = 40
GNCH = EPT // GCHUNK


def _gather_body(row_hbm, col_hbm, tr_hbm, tc_hbm, tpre_hbm, relp_hbm,
                 ridx_v, cidx_v, a0, a1, b0, b1, t0, t1, r0, r1,
                 sg0, sg1, sw0, sw1):
    c = lax.axis_index("c")
    s = lax.axis_index("s")
    base0 = c * EPC + s * EPT
    abufs = (a0, a1)
    bbufs = (b0, b1)
    tbufs = (t0, t1)
    rbufs = (r0, r1)
    gsems = (sg0, sg1)
    wsems = (sw0, sw1)

    pltpu.sync_copy(row_hbm.at[pl.ds(base0, EPT)], ridx_v)
    pltpu.sync_copy(col_hbm.at[pl.ds(base0, EPT)], cidx_v)

    def start_gather(cix, bsel):
        ri = ridx_v.at[pl.ds(cix * GCHUNK, GCHUNK)]
        ci = cidx_v.at[pl.ds(cix * GCHUNK, GCHUNK)]
        pltpu.async_copy(tr_hbm.at[ri], abufs[bsel], gsems[bsel])
        pltpu.async_copy(tc_hbm.at[ci], bbufs[bsel], gsems[bsel])

    def wait_gather(cix, bsel):
        ri = ridx_v.at[pl.ds(cix * GCHUNK, GCHUNK)]
        pltpu.make_async_copy(tr_hbm.at[ri], abufs[bsel],
                              gsems[bsel]).wait()
        pltpu.make_async_copy(tr_hbm.at[ri], bbufs[bsel],
                              gsems[bsel]).wait()

    def start_write(cix, bsel):
        base = base0 + cix * GCHUNK
        pltpu.async_copy(tbufs[bsel], tpre_hbm.at[pl.ds(base, GCHUNK), :],
                         wsems[bsel])
        pltpu.async_copy(rbufs[bsel], relp_hbm.at[pl.ds(base, GCHUNK), :],
                         wsems[bsel])

    def wait_write(bsel):
        pltpu.make_async_copy(tbufs[bsel],
                              tpre_hbm.at[pl.ds(base0, GCHUNK), :],
                              wsems[bsel]).wait()
        pltpu.make_async_copy(rbufs[bsel],
                              relp_hbm.at[pl.ds(base0, GCHUNK), :],
                              wsems[bsel]).wait()

    def compute(bsel):
        a_v, b_v, t_v, r_v = abufs[bsel], bbufs[bsel], tbufs[bsel], rbufs[bsel]

        def rowop(i, carry2):
            for l in range(D // 16):
                sl = pl.ds(l * 16, 16)
                t_v[i, sl] = a_v[i, sl] + b_v[i, sl]
            sl = pl.ds(D, 16)
            r_v[i, :] = a_v[i, sl] - b_v[i, sl]
            return carry2

        lax.fori_loop(0, GCHUNK, rowop, 0)

    def step(cix, bsel):
        wait_gather(cix, bsel)

        @pl.when(cix >= 2)
        def _():
            wait_write(bsel)

        compute(bsel)
        start_write(cix, bsel)

        @pl.when(cix + 2 < GNCH)
        def _():
            start_gather(cix + 2, bsel)

    start_gather(0, 0)
    start_gather(1, 1)

    def outer(k, carry):
        step(2 * k, 0)
        step(2 * k + 1, 1)
        return carry

    lax.fori_loop(0, GNCH // 2, outer, 0)
    wait_write(0)
    wait_write(1)


def _sc_gather(row, col, t_r, t_c):
    return pl.kernel(
        _gather_body,
        out_type=[
            jax.ShapeDtypeStruct((E, D), jnp.float32),
            jax.ShapeDtypeStruct((E, CPAD), jnp.float32),
        ],
        mesh=plsc.VectorSubcoreMesh(core_axis_name="c", subcore_axis_name="s",
                                    num_cores=SC_CORES,
                                    num_subcores=SC_TILES),
        scratch_types=[
            pltpu.VMEM((EPT,), jnp.int32),
            pltpu.VMEM((EPT,), jnp.int32),
            pltpu.VMEM((GCHUNK, TW), jnp.float32),
            pltpu.VMEM((GCHUNK, TW), jnp.float32),
            pltpu.VMEM((GCHUNK, TW), jnp.float32),
            pltpu.VMEM((GCHUNK, TW), jnp.float32),
            pltpu.VMEM((GCHUNK, D), jnp.float32),
            pltpu.VMEM((GCHUNK, D), jnp.float32),
            pltpu.VMEM((GCHUNK, CPAD), jnp.float32),
            pltpu.VMEM((GCHUNK, CPAD), jnp.float32),
            pltpu.SemaphoreType.DMA,
            pltpu.SemaphoreType.DMA,
            pltpu.SemaphoreType.DMA,
            pltpu.SemaphoreType.DMA,
        ],
    )(row, col, t_r, t_c)


def _scatter_body(row_hbm, m2_hbm, tr16_hbm, zm_hbm, outm_hbm, outc_hbm,
                  accm, ridx_v, m_v, t16_v, tw_v):
    c = lax.axis_index("c")
    s = lax.axis_index("s")
    base0 = c * EPC + s * EPT

    pltpu.sync_copy(zm_hbm.at[pl.ds(s * NPT, NPT), :],
                    accm.at[pl.ds(s * NPT, NPT), :])
    plsc.subcore_barrier()

    def loop_m(j, carry):
        base = base0 + j * CHUNK
        pltpu.sync_copy(row_hbm.at[pl.ds(base, CHUNK)], ridx_v)
        pltpu.sync_copy(m2_hbm.at[pl.ds(base, CHUNK), :], m_v)
        pltpu.sync_copy(m_v, accm.at[ridx_v], add=True)
        return carry

    lax.fori_loop(0, NCH, loop_m, 0)
    plsc.subcore_barrier()
    pltpu.sync_copy(accm.at[pl.ds(s * NPT, NPT), :],
                    outm_hbm.at[pl.ds(c * NP + s * NPT, NPT), :])
    plsc.subcore_barrier()

    pltpu.sync_copy(zm_hbm.at[pl.ds(s * NPT, NPT), :],
                    accm.at[pl.ds(s * NPT, NPT), :])
    pltpu.sync_copy(zm_hbm.at[pl.ds(0, CHUNK), :], tw_v)
    plsc.subcore_barrier()

    def loop_c(j, carry):
        base = base0 + j * CHUNK
        pltpu.sync_copy(row_hbm.at[pl.ds(base, CHUNK)], ridx_v)
        pltpu.sync_copy(tr16_hbm.at[pl.ds(base, CHUNK), :], t16_v)

        def rowop(i, carry2):
            tw_v[i, pl.ds(0, 16)] = t16_v[i, :]
            return carry2

        lax.fori_loop(0, CHUNK, rowop, 0)
        pltpu.sync_copy(tw_v, accm.at[ridx_v], add=True)
        return carry

    lax.fori_loop(0, NCH, loop_c, 0)
    plsc.subcore_barrier()
    pltpu.sync_copy(accm.at[pl.ds(s * NPT, NPT), :],
                    outc_hbm.at[pl.ds(c * NP + s * NPT, NPT), :])


def _sc_scatter(row, m2, transp, zm):
    return pl.kernel(
        _scatter_body,
        out_type=[
            jax.ShapeDtypeStruct((SC_CORES * NP, D), jnp.float32),
            jax.ShapeDtypeStruct((SC_CORES * NP, D), jnp.float32),
        ],
        mesh=plsc.VectorSubcoreMesh(core_axis_name="c", subcore_axis_name="s",
                                    num_cores=SC_CORES,
                                    num_subcores=SC_TILES),
        scratch_types=[
            pltpu.VMEM_SHARED((NP, D), jnp.float32),
            pltpu.VMEM((CHUNK,), jnp.int32),
            pltpu.VMEM((CHUNK, D), jnp.float32),
            pltpu.VMEM((CHUNK, CPAD), jnp.float32),
            pltpu.VMEM((CHUNK, D), jnp.float32),
        ],
    )(row, m2, transp, zm)


def _prep_body(af_ref, coordp_ref, atab_ref, w1a_ref, w1b_ref,
               h_ref, tr_ref, tc_ref):
    ids = af_ref[0, 0, :]
    oh = (lax.broadcast_in_dim(ids, (NODE_BLK, NTYPES), (0,))
          == lax.broadcasted_iota(jnp.int32, (NODE_BLK, NTYPES), 1)
          ).astype(jnp.float32)
    h = jnp.dot(oh, atab_ref[...], preferred_element_type=jnp.float32)
    h_ref[...] = h
    z = jnp.zeros((NODE_BLK, TW - D - CPAD), jnp.float32)
    cp = coordp_ref[...]
    tr_ref[...] = jnp.concatenate(
        [jnp.dot(h, w1a_ref[...], preferred_element_type=jnp.float32),
         cp, z], axis=1)
    tc_ref[...] = jnp.concatenate(
        [jnp.dot(h, w1b_ref[...], preferred_element_type=jnp.float32),
         cp, z], axis=1)


def _edge_body(tpre_ref, relp_ref, et_ref, eap_ref, w1d_ref, w2_ref, b2_ref,
               wc1_ref, bc1_ref, wc2_ref, bc2_ref, etab_ref,
               m2_ref, tr_ref, ea_ref):
    relp = relp_ref[...]
    d2 = jnp.sum(relp * relp, axis=1, keepdims=True)
    ids = et_ref[0, 0, :]
    oh = (lax.broadcast_in_dim(ids, (EDGE_BLK, 8), (0,))
          == lax.broadcasted_iota(jnp.int32, (EDGE_BLK, 8), 1)
          ).astype(jnp.float32)
    t1 = (tpre_ref[...] + d2 * w1d_ref[...]
          + jnp.dot(oh, eap_ref[...], preferred_element_type=jnp.float32))
    m = jax.nn.silu(t1)
    m2 = jax.nn.silu(
        jnp.dot(m, w2_ref[...], preferred_element_type=jnp.float32)
        + b2_ref[...])
    c1 = jax.nn.silu(
        jnp.dot(m2, wc1_ref[...], preferred_element_type=jnp.float32)
        + bc1_ref[...])
    cw = jnp.dot(c1, wc2_ref[...], preferred_element_type=jnp.float32) \
        + bc2_ref[...]
    m2_ref[...] = m2
    tr_ref[...] = relp * cw
    ea_ref[...] = jnp.dot(oh, etab_ref[...],
                          preferred_element_type=jnp.float32)


def _node_body(h_ref, agg0_ref, agg1_ref, wn1a_ref, wn1b_ref, bn1_ref,
               wn2_ref, bn2_ref, out_ref):
    h = h_ref[...]
    agg = agg0_ref[...] + agg1_ref[...]
    t = (jnp.dot(h, wn1a_ref[...], preferred_element_type=jnp.float32)
         + jnp.dot(agg, wn1b_ref[...], preferred_element_type=jnp.float32)
         + bn1_ref[...])
    out_ref[...] = h + jnp.dot(jax.nn.silu(t), wn2_ref[...],
                               preferred_element_type=jnp.float32) \
        + bn2_ref[...]


def _full(shape):
    return pl.BlockSpec(shape, lambda i: tuple(0 for _ in shape))


def kernel(atom_feats, coord, edge_index, edge_type_ids, atom_table,
           edge_table, W1, b1, W2, b2, Wc1, bc1, Wc2, bc2, Wn1, bn1, Wn2,
           bn2):
    row = edge_index[0]
    col = edge_index[1]

    coordp = jnp.pad(coord, ((0, 0), (0, CPAD - 3)))
    af3 = atom_feats.reshape(N // NODE_BLK, 1, NODE_BLK)
    et3 = edge_type_ids.reshape(E // EDGE_BLK, 1, EDGE_BLK)
    eap8 = jnp.pad(edge_table @ W1[2 * D + 1:] + b1, ((0, 8 - 4), (0, 0)))
    etab8 = jnp.pad(edge_table, ((0, 8 - 4), (0, 0)))
    w1d = W1[2 * D][None, :]

    h, t_r, t_c = pl.pallas_call(
        _prep_body,
        grid=(N // NODE_BLK,),
        in_specs=[
            pl.BlockSpec((1, 1, NODE_BLK), lambda i: (i, 0, 0)),
            pl.BlockSpec((NODE_BLK, CPAD), lambda i: (i, 0)),
            _full((NTYPES, D)),
            _full((D, D)),
            _full((D, D)),
        ],
        out_specs=[
            pl.BlockSpec((NODE_BLK, D), lambda i: (i, 0)),
            pl.BlockSpec((NODE_BLK, TW), lambda i: (i, 0)),
            pl.BlockSpec((NODE_BLK, TW), lambda i: (i, 0)),
        ],
        out_shape=[
            jax.ShapeDtypeStruct((N, D), jnp.float32),
            jax.ShapeDtypeStruct((N, TW), jnp.float32),
            jax.ShapeDtypeStruct((N, TW), jnp.float32),
        ],
    )(af3, coordp, atom_table, W1[:D], W1[D:2 * D])

    tpre, relp = _sc_gather(row, col, t_r, t_c)

    m2, transp, edge_attr = pl.pallas_call(
        _edge_body,
        grid=(E // EDGE_BLK,),
        in_specs=[
            pl.BlockSpec((EDGE_BLK, D), lambda i: (i, 0)),
            pl.BlockSpec((EDGE_BLK, CPAD), lambda i: (i, 0)),
            pl.BlockSpec((1, 1, EDGE_BLK), lambda i: (i, 0, 0)),
            _full((8, D)),
            _full((1, D)),
            _full((D, D)),
            _full((D,)),
            _full((D, D)),
            _full((D,)),
            _full((D, 1)),
            _full((1,)),
            _full((8, DE)),
        ],
        out_specs=[
            pl.BlockSpec((EDGE_BLK, D), lambda i: (i, 0)),
            pl.BlockSpec((EDGE_BLK, CPAD), lambda i: (i, 0)),
            pl.BlockSpec((EDGE_BLK, DE), lambda i: (i, 0)),
        ],
        out_shape=[
            jax.ShapeDtypeStruct((E, D), jnp.float32),
            jax.ShapeDtypeStruct((E, CPAD), jnp.float32),
            jax.ShapeDtypeStruct((E, DE), jnp.float32),
        ],
    )(tpre, relp, et3, eap8, w1d, W2, b2, Wc1, bc1, Wc2, bc2, etab8)

    zm = jnp.zeros((NP, D), jnp.float32)
    outm, outc = _sc_scatter(row, m2, transp, zm)
    aggm = outm.reshape(SC_CORES, NP, D)
    aggc = outc.reshape(SC_CORES, NP, D)
    coord_out = coord + (aggc[0, :N, :3] + aggc[1, :N, :3]) / C_NORM

    h_out = pl.pallas_call(
        _node_body,
        grid=(N // NODE_BLK,),
        in_specs=[
            pl.BlockSpec((NODE_BLK, D), lambda i: (i, 0)),
            pl.BlockSpec((NODE_BLK, D), lambda i: (i, 0)),
            pl.BlockSpec((NODE_BLK, D), lambda i: (i, 0)),
            _full((D, D)),
            _full((D, D)),
            _full((D,)),
            _full((D, D)),
            _full((D,)),
        ],
        out_specs=pl.BlockSpec((NODE_BLK, D), lambda i: (i, 0)),
        out_shape=jax.ShapeDtypeStruct((N, D), jnp.float32),
    )(h, aggm[0, :N], aggm[1, :N], Wn1[:D], Wn1[D:], bn1, Wn2, bn2)

    return (h_out, coord_out, edge_attr)


# speedup vs baseline: 4.8305x; 1.1929x over previous
import jax
import jax.numpy as jnp
from jax import lax
from jax.experimental import pallas as pl
from jax.experimental.pallas import tpu as pltpu
from jax.experimental.pallas import tpu_sc as plsc

N = 10000
E = 320000
D = 128
DE = 16
C_NORM = 32.0
NTYPES = 16

EDGE_BLK = 2560
NODE_BLK = 2000
TW = 2 * D
CPAD = 16

SC_CORES = 2
SC_TILES = 16
CHUNK = 80
EPC = E // SC_CORES
EPT = EPC // SC_TILES
NCH = EPT // CHUNK
NP = 10240
NPT = NP // SC_TILES
GCHUNK = 40
GNCH = EPT // GCHUNK


def _gather_body(row_hbm, col_hbm, tr_hbm, tc_hbm, tpre_hbm, relp_hbm,
                 ridx_v, cidx_v, a0, a1, b0, b1, t0, t1, r0, r1,
                 sg0, sg1, sw0, sw1):
    c = lax.axis_index("c")
    s = lax.axis_index("s")
    base0 = c * EPC + s * EPT
    abufs = (a0, a1)
    bbufs = (b0, b1)
    tbufs = (t0, t1)
    rbufs = (r0, r1)
    gsems = (sg0, sg1)
    wsems = (sw0, sw1)

    pltpu.sync_copy(row_hbm.at[pl.ds(base0, EPT)], ridx_v)
    pltpu.sync_copy(col_hbm.at[pl.ds(base0, EPT)], cidx_v)

    def start_gather(cix, bsel):
        ri = ridx_v.at[pl.ds(cix * GCHUNK, GCHUNK)]
        ci = cidx_v.at[pl.ds(cix * GCHUNK, GCHUNK)]
        pltpu.async_copy(tr_hbm.at[ri], abufs[bsel], gsems[bsel])
        pltpu.async_copy(tc_hbm.at[ci], bbufs[bsel], gsems[bsel])

    def wait_gather(cix, bsel):
        ri = ridx_v.at[pl.ds(cix * GCHUNK, GCHUNK)]
        pltpu.make_async_copy(tr_hbm.at[ri], abufs[bsel],
                              gsems[bsel]).wait()
        pltpu.make_async_copy(tr_hbm.at[ri], bbufs[bsel],
                              gsems[bsel]).wait()

    def start_write(cix, bsel):
        base = base0 + cix * GCHUNK
        pltpu.async_copy(tbufs[bsel], tpre_hbm.at[pl.ds(base, GCHUNK), :],
                         wsems[bsel])
        pltpu.async_copy(rbufs[bsel], relp_hbm.at[pl.ds(base, GCHUNK), :],
                         wsems[bsel])

    def wait_write(bsel):
        pltpu.make_async_copy(tbufs[bsel],
                              tpre_hbm.at[pl.ds(base0, GCHUNK), :],
                              wsems[bsel]).wait()
        pltpu.make_async_copy(rbufs[bsel],
                              relp_hbm.at[pl.ds(base0, GCHUNK), :],
                              wsems[bsel]).wait()

    def compute(bsel):
        a_v, b_v, t_v, r_v = abufs[bsel], bbufs[bsel], tbufs[bsel], rbufs[bsel]

        def rowop(i, carry2):
            for l in range(D // 16):
                sl = pl.ds(l * 16, 16)
                t_v[i, sl] = a_v[i, sl] + b_v[i, sl]
            sl = pl.ds(D, 16)
            r_v[i, :] = a_v[i, sl] - b_v[i, sl]
            return carry2

        lax.fori_loop(0, GCHUNK, rowop, 0)

    def step(cix, bsel):
        wait_gather(cix, bsel)

        @pl.when(cix >= 2)
        def _():
            wait_write(bsel)

        compute(bsel)
        start_write(cix, bsel)

        @pl.when(cix + 2 < GNCH)
        def _():
            start_gather(cix + 2, bsel)

    start_gather(0, 0)
    start_gather(1, 1)

    def outer(k, carry):
        step(2 * k, 0)
        step(2 * k + 1, 1)
        return carry

    lax.fori_loop(0, GNCH // 2, outer, 0)
    wait_write(0)
    wait_write(1)


def _sc_gather(row, col, t_r, t_c):
    return pl.kernel(
        _gather_body,
        out_type=[
            jax.ShapeDtypeStruct((E, D), jnp.float32),
            jax.ShapeDtypeStruct((E, CPAD), jnp.float32),
        ],
        mesh=plsc.VectorSubcoreMesh(core_axis_name="c", subcore_axis_name="s",
                                    num_cores=SC_CORES,
                                    num_subcores=SC_TILES),
        scratch_types=[
            pltpu.VMEM((EPT,), jnp.int32),
            pltpu.VMEM((EPT,), jnp.int32),
            pltpu.VMEM((GCHUNK, TW), jnp.float32),
            pltpu.VMEM((GCHUNK, TW), jnp.float32),
            pltpu.VMEM((GCHUNK, TW), jnp.float32),
            pltpu.VMEM((GCHUNK, TW), jnp.float32),
            pltpu.VMEM((GCHUNK, D), jnp.float32),
            pltpu.VMEM((GCHUNK, D), jnp.float32),
            pltpu.VMEM((GCHUNK, CPAD), jnp.float32),
            pltpu.VMEM((GCHUNK, CPAD), jnp.float32),
            pltpu.SemaphoreType.DMA,
            pltpu.SemaphoreType.DMA,
            pltpu.SemaphoreType.DMA,
            pltpu.SemaphoreType.DMA,
        ],
    )(row, col, t_r, t_c)


def _scatter_body(idx3_hbm, m2_hbm, tr16_hbm, zm_hbm, outm_hbm, outc_hbm,
                  accm, idx_v, m0, m1, t0, sl0, sl1):
    c = lax.axis_index("c")
    s = lax.axis_index("s")
    w = c * SC_TILES + s
    base0 = c * EPC + s * EPT
    mbufs = (m0, m1)
    lsems = (sl0, sl1)

    pltpu.sync_copy(idx3_hbm.at[w], idx_v)

    pltpu.sync_copy(zm_hbm.at[pl.ds(s * NPT, NPT), :],
                    accm.at[pl.ds(s * NPT, NPT), :])
    plsc.subcore_barrier()

    def start_load_m(j, bsel):
        base = base0 + j * CHUNK
        pltpu.async_copy(m2_hbm.at[pl.ds(base, CHUNK), :], mbufs[bsel],
                         lsems[bsel])

    def step_m(j, bsel):
        pltpu.make_async_copy(m2_hbm.at[pl.ds(base0, CHUNK), :],
                              mbufs[bsel], lsems[bsel]).wait()
        pltpu.sync_copy(mbufs[bsel], accm.at[idx_v.at[j]], add=True)

        @pl.when(j + 2 < NCH)
        def _():
            start_load_m(j + 2, bsel)

    start_load_m(0, 0)
    start_load_m(1, 1)

    def loop_m(k, carry):
        step_m(2 * k, 0)
        step_m(2 * k + 1, 1)
        return carry

    lax.fori_loop(0, NCH // 2, loop_m, 0)
    step_m(NCH - 1, 0)
    plsc.subcore_barrier()
    pltpu.sync_copy(accm.at[pl.ds(s * NPT, NPT), :],
                    outm_hbm.at[pl.ds(c * NP + s * NPT, NPT), :])
    plsc.subcore_barrier()

    pltpu.sync_copy(zm_hbm.at[pl.ds(s * NPT, NPT), :],
                    accm.at[pl.ds(s * NPT, NPT), :])
    pltpu.sync_copy(zm_hbm.at[pl.ds(0, CHUNK), :], m0)
    pltpu.sync_copy(zm_hbm.at[pl.ds(0, CHUNK), :], m1)
    plsc.subcore_barrier()

    def start_load_t(j):
        base = base0 + j * CHUNK
        pltpu.async_copy(tr16_hbm.at[pl.ds(base, CHUNK), :], t0, sl0)

    def step_t(j, bsel):
        pltpu.make_async_copy(tr16_hbm.at[pl.ds(base0, CHUNK), :],
                              t0, sl0).wait()
        tw_v = mbufs[bsel]

        def rowop(i, carry2):
            tw_v[i, pl.ds(0, 16)] = t0[i, :]
            return carry2

        lax.fori_loop(0, CHUNK, rowop, 0)

        @pl.when(j + 1 < NCH)
        def _():
            start_load_t(j + 1)

        pltpu.sync_copy(tw_v, accm.at[idx_v.at[j]], add=True)

    start_load_t(0)

    def loop_t(k, carry):
        step_t(2 * k, 0)
        step_t(2 * k + 1, 1)
        return carry

    lax.fori_loop(0, NCH // 2, loop_t, 0)
    step_t(NCH - 1, 0)
    plsc.subcore_barrier()
    pltpu.sync_copy(accm.at[pl.ds(s * NPT, NPT), :],
                    outc_hbm.at[pl.ds(c * NP + s * NPT, NPT), :])


def _sc_scatter(row, m2, transp, zm):
    idx3 = row.reshape(SC_CORES * SC_TILES, NCH, CHUNK)
    return pl.kernel(
        _scatter_body,
        out_type=[
            jax.ShapeDtypeStruct((SC_CORES * NP, D), jnp.float32),
            jax.ShapeDtypeStruct((SC_CORES * NP, D), jnp.float32),
        ],
        mesh=plsc.VectorSubcoreMesh(core_axis_name="c", subcore_axis_name="s",
                                    num_cores=SC_CORES,
                                    num_subcores=SC_TILES),
        scratch_types=[
            pltpu.VMEM_SHARED((NP, D), jnp.float32),
            pltpu.VMEM((NCH, CHUNK), jnp.int32),
            pltpu.VMEM((CHUNK, D), jnp.float32),
            pltpu.VMEM((CHUNK, D), jnp.float32),
            pltpu.VMEM((CHUNK, CPAD), jnp.float32),
            pltpu.SemaphoreType.DMA,
            pltpu.SemaphoreType.DMA,
        ],
    )(idx3, m2, transp, zm)


def _prep_body(af_ref, coordp_ref, atab_ref, w1a_ref, w1b_ref,
               h_ref, tr_ref, tc_ref):
    ids = af_ref[0, 0, :]
    oh = (lax.broadcast_in_dim(ids, (NODE_BLK, NTYPES), (0,))
          == lax.broadcasted_iota(jnp.int32, (NODE_BLK, NTYPES), 1)
          ).astype(jnp.float32)
    h = jnp.dot(oh, atab_ref[...], preferred_element_type=jnp.float32)
    h_ref[...] = h
    z = jnp.zeros((NODE_BLK, TW - D - CPAD), jnp.float32)
    cp = coordp_ref[...]
    tr_ref[...] = jnp.concatenate(
        [jnp.dot(h, w1a_ref[...], preferred_element_type=jnp.float32),
         cp, z], axis=1)
    tc_ref[...] = jnp.concatenate(
        [jnp.dot(h, w1b_ref[...], preferred_element_type=jnp.float32),
         cp, z], axis=1)


def _edge_body(tpre_ref, relp_ref, et_ref, eap_ref, w1d_ref, w2_ref, b2_ref,
               wc1_ref, bc1_ref, wc2_ref, bc2_ref, etab_ref,
               m2_ref, tr_ref, ea_ref):
    relp = relp_ref[...]
    d2 = jnp.sum(relp * relp, axis=1, keepdims=True)
    ids = et_ref[0, 0, :]
    oh = (lax.broadcast_in_dim(ids, (EDGE_BLK, 8), (0,))
          == lax.broadcasted_iota(jnp.int32, (EDGE_BLK, 8), 1)
          ).astype(jnp.float32)
    t1 = (tpre_ref[...] + d2 * w1d_ref[...]
          + jnp.dot(oh, eap_ref[...], preferred_element_type=jnp.float32))
    m = jax.nn.silu(t1)
    m2 = jax.nn.silu(
        jnp.dot(m, w2_ref[...], preferred_element_type=jnp.float32)
        + b2_ref[...])
    c1 = jax.nn.silu(
        jnp.dot(m2, wc1_ref[...], preferred_element_type=jnp.float32)
        + bc1_ref[...])
    cw = jnp.dot(c1, wc2_ref[...], preferred_element_type=jnp.float32) \
        + bc2_ref[...]
    m2_ref[...] = m2
    tr_ref[...] = relp * cw
    ea_ref[...] = jnp.dot(oh, etab_ref[...],
                          preferred_element_type=jnp.float32)


def _node_body(h_ref, agg0_ref, agg1_ref, wn1a_ref, wn1b_ref, bn1_ref,
               wn2_ref, bn2_ref, out_ref):
    h = h_ref[...]
    agg = agg0_ref[...] + agg1_ref[...]
    t = (jnp.dot(h, wn1a_ref[...], preferred_element_type=jnp.float32)
         + jnp.dot(agg, wn1b_ref[...], preferred_element_type=jnp.float32)
         + bn1_ref[...])
    out_ref[...] = h + jnp.dot(jax.nn.silu(t), wn2_ref[...],
                               preferred_element_type=jnp.float32) \
        + bn2_ref[...]


def _full(shape):
    return pl.BlockSpec(shape, lambda i: tuple(0 for _ in shape))


def kernel(atom_feats, coord, edge_index, edge_type_ids, atom_table,
           edge_table, W1, b1, W2, b2, Wc1, bc1, Wc2, bc2, Wn1, bn1, Wn2,
           bn2):
    row = edge_index[0]
    col = edge_index[1]

    coordp = jnp.pad(coord, ((0, 0), (0, CPAD - 3)))
    af3 = atom_feats.reshape(N // NODE_BLK, 1, NODE_BLK)
    et3 = edge_type_ids.reshape(E // EDGE_BLK, 1, EDGE_BLK)
    eap8 = jnp.pad(edge_table @ W1[2 * D + 1:] + b1, ((0, 8 - 4), (0, 0)))
    etab8 = jnp.pad(edge_table, ((0, 8 - 4), (0, 0)))
    w1d = W1[2 * D][None, :]

    h, t_r, t_c = pl.pallas_call(
        _prep_body,
        grid=(N // NODE_BLK,),
        in_specs=[
            pl.BlockSpec((1, 1, NODE_BLK), lambda i: (i, 0, 0)),
            pl.BlockSpec((NODE_BLK, CPAD), lambda i: (i, 0)),
            _full((NTYPES, D)),
            _full((D, D)),
            _full((D, D)),
        ],
        out_specs=[
            pl.BlockSpec((NODE_BLK, D), lambda i: (i, 0)),
            pl.BlockSpec((NODE_BLK, TW), lambda i: (i, 0)),
            pl.BlockSpec((NODE_BLK, TW), lambda i: (i, 0)),
        ],
        out_shape=[
            jax.ShapeDtypeStruct((N, D), jnp.float32),
            jax.ShapeDtypeStruct((N, TW), jnp.float32),
            jax.ShapeDtypeStruct((N, TW), jnp.float32),
        ],
    )(af3, coordp, atom_table, W1[:D], W1[D:2 * D])

    tpre, relp = _sc_gather(row, col, t_r, t_c)

    m2, transp, edge_attr = pl.pallas_call(
        _edge_body,
        grid=(E // EDGE_BLK,),
        in_specs=[
            pl.BlockSpec((EDGE_BLK, D), lambda i: (i, 0)),
            pl.BlockSpec((EDGE_BLK, CPAD), lambda i: (i, 0)),
            pl.BlockSpec((1, 1, EDGE_BLK), lambda i: (i, 0, 0)),
            _full((8, D)),
            _full((1, D)),
            _full((D, D)),
            _full((D,)),
            _full((D, D)),
            _full((D,)),
            _full((D, 1)),
            _full((1,)),
            _full((8, DE)),
        ],
        out_specs=[
            pl.BlockSpec((EDGE_BLK, D), lambda i: (i, 0)),
            pl.BlockSpec((EDGE_BLK, CPAD), lambda i: (i, 0)),
            pl.BlockSpec((EDGE_BLK, DE), lambda i: (i, 0)),
        ],
        out_shape=[
            jax.ShapeDtypeStruct((E, D), jnp.float32),
            jax.ShapeDtypeStruct((E, CPAD), jnp.float32),
            jax.ShapeDtypeStruct((E, DE), jnp.float32),
        ],
    )(tpre, relp, et3, eap8, w1d, W2, b2, Wc1, bc1, Wc2, bc2, etab8)

    zm = jnp.zeros((NP, D), jnp.float32)
    outm, outc = _sc_scatter(row, m2, transp, zm)
    aggm = outm.reshape(SC_CORES, NP, D)
    aggc = outc.reshape(SC_CORES, NP, D)
    coord_out = coord + (aggc[0, :N, :3] + aggc[1, :N, :3]) / C_NORM

    h_out = pl.pallas_call(
        _node_body,
        grid=(N // NODE_BLK,),
        in_specs=[
            pl.BlockSpec((NODE_BLK, D), lambda i: (i, 0)),
            pl.BlockSpec((NODE_BLK, D), lambda i: (i, 0)),
            pl.BlockSpec((NODE_BLK, D), lambda i: (i, 0)),
            _full((D, D)),
            _full((D, D)),
            _full((D,)),
            _full((D, D)),
            _full((D,)),
        ],
        out_specs=pl.BlockSpec((NODE_BLK, D), lambda i: (i, 0)),
        out_shape=jax.ShapeDtypeStruct((N, D), jnp.float32),
    )(h, aggm[0, :N], aggm[1, :N], Wn1[:D], Wn1[D:], bn1, Wn2, bn2)

    return (h_out, coord_out, edge_attr)


# speedup vs baseline: 4.8363x; 1.0012x over previous
import jax
import jax.numpy as jnp
from jax import lax
from jax.experimental import pallas as pl
from jax.experimental.pallas import tpu as pltpu
from jax.experimental.pallas import tpu_sc as plsc

N = 10000
E = 320000
D = 128
DE = 16
C_NORM = 32.0
NTYPES = 16

EDGE_BLK = 2560
NODE_BLK = 2000
TW = 2 * D
CPAD = 16

SC_CORES = 2
SC_TILES = 16
CHUNK = 80
EPC = E // SC_CORES
EPT = EPC // SC_TILES
NCH = EPT // CHUNK
NP = 10240
NPT = NP // SC_TILES
GCHUNK = 40
GNCH = EPT // GCHUNK


def _gather_body(row_hbm, col_hbm, tr_hbm, tc_hbm, tpre_hbm, relp_hbm,
                 ridx_v, cidx_v, a0, a1, a2, b0, b1, b2, t0, t1, t2,
                 r0, r1, r2, sg0, sg1, sg2, sw0, sw1, sw2):
    c = lax.axis_index("c")
    s = lax.axis_index("s")
    base0 = c * EPC + s * EPT
    abufs = (a0, a1, a2)
    bbufs = (b0, b1, b2)
    tbufs = (t0, t1, t2)
    rbufs = (r0, r1, r2)
    gsems = (sg0, sg1, sg2)
    wsems = (sw0, sw1, sw2)

    pltpu.sync_copy(row_hbm.at[pl.ds(base0, EPT)], ridx_v)
    pltpu.sync_copy(col_hbm.at[pl.ds(base0, EPT)], cidx_v)

    def start_gather(cix, bsel):
        ri = ridx_v.at[pl.ds(cix * GCHUNK, GCHUNK)]
        ci = cidx_v.at[pl.ds(cix * GCHUNK, GCHUNK)]
        pltpu.async_copy(tr_hbm.at[ri], abufs[bsel], gsems[bsel])
        pltpu.async_copy(tc_hbm.at[ci], bbufs[bsel], gsems[bsel])

    def wait_gather(cix, bsel):
        ri = ridx_v.at[pl.ds(cix * GCHUNK, GCHUNK)]
        pltpu.make_async_copy(tr_hbm.at[ri], abufs[bsel],
                              gsems[bsel]).wait()
        pltpu.make_async_copy(tr_hbm.at[ri], bbufs[bsel],
                              gsems[bsel]).wait()

    def start_write(cix, bsel):
        base = base0 + cix * GCHUNK
        pltpu.async_copy(tbufs[bsel], tpre_hbm.at[pl.ds(base, GCHUNK), :],
                         wsems[bsel])
        pltpu.async_copy(rbufs[bsel], relp_hbm.at[pl.ds(base, GCHUNK), :],
                         wsems[bsel])

    def wait_write(bsel):
        pltpu.make_async_copy(tbufs[bsel],
                              tpre_hbm.at[pl.ds(base0, GCHUNK), :],
                              wsems[bsel]).wait()
        pltpu.make_async_copy(rbufs[bsel],
                              relp_hbm.at[pl.ds(base0, GCHUNK), :],
                              wsems[bsel]).wait()

    def compute(bsel):
        a_v, b_v, t_v, r_v = abufs[bsel], bbufs[bsel], tbufs[bsel], rbufs[bsel]

        def rowop(i, carry2):
            for l in range(D // 16):
                sl = pl.ds(l * 16, 16)
                t_v[i, sl] = a_v[i, sl] + b_v[i, sl]
            sl = pl.ds(D, 16)
            r_v[i, :] = a_v[i, sl] - b_v[i, sl]
            return carry2

        lax.fori_loop(0, GCHUNK, rowop, 0)

    def step(cix, bsel):
        wait_gather(cix, bsel)

        @pl.when(cix >= 3)
        def _():
            wait_write(bsel)

        compute(bsel)
        start_write(cix, bsel)

        @pl.when(cix + 3 < GNCH)
        def _():
            start_gather(cix + 3, bsel)

    start_gather(0, 0)
    start_gather(1, 1)
    start_gather(2, 2)

    def outer(k, carry):
        step(3 * k, 0)
        step(3 * k + 1, 1)
        step(3 * k + 2, 2)
        return carry

    lax.fori_loop(0, GNCH // 3, outer, 0)
    step(GNCH - 1, 0)
    wait_write(0)
    wait_write(1)
    wait_write(2)


def _sc_gather(row, col, t_r, t_c):
    return pl.kernel(
        _gather_body,
        out_type=[
            jax.ShapeDtypeStruct((E, D), jnp.float32),
            jax.ShapeDtypeStruct((E, CPAD), jnp.float32),
        ],
        mesh=plsc.VectorSubcoreMesh(core_axis_name="c", subcore_axis_name="s",
                                    num_cores=SC_CORES,
                                    num_subcores=SC_TILES),
        scratch_types=[
            pltpu.VMEM((EPT,), jnp.int32),
            pltpu.VMEM((EPT,), jnp.int32),
            pltpu.VMEM((GCHUNK, TW), jnp.float32),
            pltpu.VMEM((GCHUNK, TW), jnp.float32),
            pltpu.VMEM((GCHUNK, TW), jnp.float32),
            pltpu.VMEM((GCHUNK, TW), jnp.float32),
            pltpu.VMEM((GCHUNK, TW), jnp.float32),
            pltpu.VMEM((GCHUNK, TW), jnp.float32),
            pltpu.VMEM((GCHUNK, D), jnp.float32),
            pltpu.VMEM((GCHUNK, D), jnp.float32),
            pltpu.VMEM((GCHUNK, D), jnp.float32),
            pltpu.VMEM((GCHUNK, CPAD), jnp.float32),
            pltpu.VMEM((GCHUNK, CPAD), jnp.float32),
            pltpu.VMEM((GCHUNK, CPAD), jnp.float32),
            pltpu.SemaphoreType.DMA,
            pltpu.SemaphoreType.DMA,
            pltpu.SemaphoreType.DMA,
            pltpu.SemaphoreType.DMA,
            pltpu.SemaphoreType.DMA,
            pltpu.SemaphoreType.DMA,
        ],
    )(row, col, t_r, t_c)


def _scatter_body(idx3_hbm, m2_hbm, tr16_hbm, zm_hbm, outm_hbm, outc_hbm,
                  accm, idx_v, m0, m1, t0, sl0, sl1):
    c = lax.axis_index("c")
    s = lax.axis_index("s")
    w = c * SC_TILES + s
    base0 = c * EPC + s * EPT
    mbufs = (m0, m1)
    lsems = (sl0, sl1)

    pltpu.sync_copy(idx3_hbm.at[w], idx_v)

    pltpu.sync_copy(zm_hbm.at[pl.ds(s * NPT, NPT), :],
                    accm.at[pl.ds(s * NPT, NPT), :])
    plsc.subcore_barrier()

    def start_load_m(j, bsel):
        base = base0 + j * CHUNK
        pltpu.async_copy(m2_hbm.at[pl.ds(base, CHUNK), :], mbufs[bsel],
                         lsems[bsel])

    def step_m(j, bsel):
        pltpu.make_async_copy(m2_hbm.at[pl.ds(base0, CHUNK), :],
                              mbufs[bsel], lsems[bsel]).wait()
        pltpu.sync_copy(mbufs[bsel], accm.at[idx_v.at[j]], add=True)

        @pl.when(j + 2 < NCH)
        def _():
            start_load_m(j + 2, bsel)

    start_load_m(0, 0)
    start_load_m(1, 1)

    def loop_m(k, carry):
        step_m(2 * k, 0)
        step_m(2 * k + 1, 1)
        return carry

    lax.fori_loop(0, NCH // 2, loop_m, 0)
    step_m(NCH - 1, 0)
    plsc.subcore_barrier()
    pltpu.sync_copy(accm.at[pl.ds(s * NPT, NPT), :],
                    outm_hbm.at[pl.ds(c * NP + s * NPT, NPT), :])
    plsc.subcore_barrier()

    pltpu.sync_copy(zm_hbm.at[pl.ds(s * NPT, NPT), :],
                    accm.at[pl.ds(s * NPT, NPT), :])
    pltpu.sync_copy(zm_hbm.at[pl.ds(0, CHUNK), :], m0)
    pltpu.sync_copy(zm_hbm.at[pl.ds(0, CHUNK), :], m1)
    plsc.subcore_barrier()

    def start_load_t(j):
        base = base0 + j * CHUNK
        pltpu.async_copy(tr16_hbm.at[pl.ds(base, CHUNK), :], t0, sl0)

    def step_t(j, bsel):
        pltpu.make_async_copy(tr16_hbm.at[pl.ds(base0, CHUNK), :],
                              t0, sl0).wait()
        tw_v = mbufs[bsel]

        def rowop(i, carry2):
            tw_v[i, pl.ds(0, 16)] = t0[i, :]
            return carry2

        lax.fori_loop(0, CHUNK, rowop, 0)

        @pl.when(j + 1 < NCH)
        def _():
            start_load_t(j + 1)

        pltpu.sync_copy(tw_v, accm.at[idx_v.at[j]], add=True)

    start_load_t(0)

    def loop_t(k, carry):
        step_t(2 * k, 0)
        step_t(2 * k + 1, 1)
        return carry

    lax.fori_loop(0, NCH // 2, loop_t, 0)
    step_t(NCH - 1, 0)
    plsc.subcore_barrier()
    pltpu.sync_copy(accm.at[pl.ds(s * NPT, NPT), :],
                    outc_hbm.at[pl.ds(c * NP + s * NPT, NPT), :])


def _sc_scatter(row, m2, transp, zm):
    idx3 = row.reshape(SC_CORES * SC_TILES, NCH, CHUNK)
    return pl.kernel(
        _scatter_body,
        out_type=[
            jax.ShapeDtypeStruct((SC_CORES * NP, D), jnp.float32),
            jax.ShapeDtypeStruct((SC_CORES * NP, D), jnp.float32),
        ],
        mesh=plsc.VectorSubcoreMesh(core_axis_name="c", subcore_axis_name="s",
                                    num_cores=SC_CORES,
                                    num_subcores=SC_TILES),
        scratch_types=[
            pltpu.VMEM_SHARED((NP, D), jnp.float32),
            pltpu.VMEM((NCH, CHUNK), jnp.int32),
            pltpu.VMEM((CHUNK, D), jnp.float32),
            pltpu.VMEM((CHUNK, D), jnp.float32),
            pltpu.VMEM((CHUNK, CPAD), jnp.float32),
            pltpu.SemaphoreType.DMA,
            pltpu.SemaphoreType.DMA,
        ],
    )(idx3, m2, transp, zm)


def _prep_body(af_ref, coordp_ref, atab_ref, w1a_ref, w1b_ref,
               h_ref, tr_ref, tc_ref):
    ids = af_ref[0, 0, :]
    oh = (lax.broadcast_in_dim(ids, (NODE_BLK, NTYPES), (0,))
          == lax.broadcasted_iota(jnp.int32, (NODE_BLK, NTYPES), 1)
          ).astype(jnp.float32)
    h = jnp.dot(oh, atab_ref[...], preferred_element_type=jnp.float32)
    h_ref[...] = h
    z = jnp.zeros((NODE_BLK, TW - D - CPAD), jnp.float32)
    cp = coordp_ref[...]
    tr_ref[...] = jnp.concatenate(
        [jnp.dot(h, w1a_ref[...], preferred_element_type=jnp.float32),
         cp, z], axis=1)
    tc_ref[...] = jnp.concatenate(
        [jnp.dot(h, w1b_ref[...], preferred_element_type=jnp.float32),
         cp, z], axis=1)


def _edge_body(tpre_ref, relp_ref, et_ref, eap_ref, w1d_ref, w2_ref, b2_ref,
               wc1_ref, bc1_ref, wc2_ref, bc2_ref, etab_ref,
               m2_ref, tr_ref, ea_ref):
    relp = relp_ref[...]
    d2 = jnp.sum(relp * relp, axis=1, keepdims=True)
    ids = et_ref[0, 0, :]
    oh = (lax.broadcast_in_dim(ids, (EDGE_BLK, 8), (0,))
          == lax.broadcasted_iota(jnp.int32, (EDGE_BLK, 8), 1)
          ).astype(jnp.float32)
    t1 = (tpre_ref[...] + d2 * w1d_ref[...]
          + jnp.dot(oh, eap_ref[...], preferred_element_type=jnp.float32))
    m = jax.nn.silu(t1)
    m2 = jax.nn.silu(
        jnp.dot(m, w2_ref[...], preferred_element_type=jnp.float32)
        + b2_ref[...])
    c1 = jax.nn.silu(
        jnp.dot(m2, wc1_ref[...], preferred_element_type=jnp.float32)
        + bc1_ref[...])
    cw = jnp.dot(c1, wc2_ref[...], preferred_element_type=jnp.float32) \
        + bc2_ref[...]
    m2_ref[...] = m2
    tr_ref[...] = relp * cw
    ea_ref[...] = jnp.dot(oh, etab_ref[...],
                          preferred_element_type=jnp.float32)


def _node_body(h_ref, agg0_ref, agg1_ref, wn1a_ref, wn1b_ref, bn1_ref,
               wn2_ref, bn2_ref, out_ref):
    h = h_ref[...]
    agg = agg0_ref[...] + agg1_ref[...]
    t = (jnp.dot(h, wn1a_ref[...], preferred_element_type=jnp.float32)
         + jnp.dot(agg, wn1b_ref[...], preferred_element_type=jnp.float32)
         + bn1_ref[...])
    out_ref[...] = h + jnp.dot(jax.nn.silu(t), wn2_ref[...],
                               preferred_element_type=jnp.float32) \
        + bn2_ref[...]


def _full(shape):
    return pl.BlockSpec(shape, lambda i: tuple(0 for _ in shape))


def kernel(atom_feats, coord, edge_index, edge_type_ids, atom_table,
           edge_table, W1, b1, W2, b2, Wc1, bc1, Wc2, bc2, Wn1, bn1, Wn2,
           bn2):
    row = edge_index[0]
    col = edge_index[1]

    coordp = jnp.pad(coord, ((0, 0), (0, CPAD - 3)))
    af3 = atom_feats.reshape(N // NODE_BLK, 1, NODE_BLK)
    et3 = edge_type_ids.reshape(E // EDGE_BLK, 1, EDGE_BLK)
    eap8 = jnp.pad(edge_table @ W1[2 * D + 1:] + b1, ((0, 8 - 4), (0, 0)))
    etab8 = jnp.pad(edge_table, ((0, 8 - 4), (0, 0)))
    w1d = W1[2 * D][None, :]

    h, t_r, t_c = pl.pallas_call(
        _prep_body,
        grid=(N // NODE_BLK,),
        in_specs=[
            pl.BlockSpec((1, 1, NODE_BLK), lambda i: (i, 0, 0)),
            pl.BlockSpec((NODE_BLK, CPAD), lambda i: (i, 0)),
            _full((NTYPES, D)),
            _full((D, D)),
            _full((D, D)),
        ],
        out_specs=[
            pl.BlockSpec((NODE_BLK, D), lambda i: (i, 0)),
            pl.BlockSpec((NODE_BLK, TW), lambda i: (i, 0)),
            pl.BlockSpec((NODE_BLK, TW), lambda i: (i, 0)),
        ],
        out_shape=[
            jax.ShapeDtypeStruct((N, D), jnp.float32),
            jax.ShapeDtypeStruct((N, TW), jnp.float32),
            jax.ShapeDtypeStruct((N, TW), jnp.float32),
        ],
    )(af3, coordp, atom_table, W1[:D], W1[D:2 * D])

    tpre, relp = _sc_gather(row, col, t_r, t_c)

    m2, transp, edge_attr = pl.pallas_call(
        _edge_body,
        grid=(E // EDGE_BLK,),
        in_specs=[
            pl.BlockSpec((EDGE_BLK, D), lambda i: (i, 0)),
            pl.BlockSpec((EDGE_BLK, CPAD), lambda i: (i, 0)),
            pl.BlockSpec((1, 1, EDGE_BLK), lambda i: (i, 0, 0)),
            _full((8, D)),
            _full((1, D)),
            _full((D, D)),
            _full((D,)),
            _full((D, D)),
            _full((D,)),
            _full((D, 1)),
            _full((1,)),
            _full((8, DE)),
        ],
        out_specs=[
            pl.BlockSpec((EDGE_BLK, D), lambda i: (i, 0)),
            pl.BlockSpec((EDGE_BLK, CPAD), lambda i: (i, 0)),
            pl.BlockSpec((EDGE_BLK, DE), lambda i: (i, 0)),
        ],
        out_shape=[
            jax.ShapeDtypeStruct((E, D), jnp.float32),
            jax.ShapeDtypeStruct((E, CPAD), jnp.float32),
            jax.ShapeDtypeStruct((E, DE), jnp.float32),
        ],
    )(tpre, relp, et3, eap8, w1d, W2, b2, Wc1, bc1, Wc2, bc2, etab8)

    zm = jnp.zeros((NP, D), jnp.float32)
    outm, outc = _sc_scatter(row, m2, transp, zm)
    aggm = outm.reshape(SC_CORES, NP, D)
    aggc = outc.reshape(SC_CORES, NP, D)
    coord_out = coord + (aggc[0, :N, :3] + aggc[1, :N, :3]) / C_NORM

    h_out = pl.pallas_call(
        _node_body,
        grid=(N // NODE_BLK,),
        in_specs=[
            pl.BlockSpec((NODE_BLK, D), lambda i: (i, 0)),
            pl.BlockSpec((NODE_BLK, D), lambda i: (i, 0)),
            pl.BlockSpec((NODE_BLK, D), lambda i: (i, 0)),
            _full((D, D)),
            _full((D, D)),
            _full((D,)),
            _full((D, D)),
            _full((D,)),
        ],
        out_specs=pl.BlockSpec((NODE_BLK, D), lambda i: (i, 0)),
        out_shape=jax.ShapeDtypeStruct((N, D), jnp.float32),
    )(h, aggm[0, :N], aggm[1, :N], Wn1[:D], Wn1[D:], bn1, Wn2, bn2)

    return (h_out, coord_out, edge_attr)


# speedup vs baseline: 4.8400x; 1.0008x over previous
import jax
import jax.numpy as jnp
from jax import lax
from jax.experimental import pallas as pl
from jax.experimental.pallas import tpu as pltpu
from jax.experimental.pallas import tpu_sc as plsc

N = 10000
E = 320000
D = 128
DE = 16
C_NORM = 32.0
NTYPES = 16

EDGE_BLK = 2560
NODE_BLK = 2000
TW = 2 * D
CPAD = 16
RW = 32

SC_CORES = 2
SC_TILES = 16
CHUNK = 80
EPC = E // SC_CORES
EPT = EPC // SC_TILES
NCH = EPT // CHUNK
NP = 10240
NPT = NP // SC_TILES
GCHUNK = 40
GNCH = EPT // GCHUNK
TWH = TW // 2
DH = D // 2
RWH = RW // 2


def _gather_body(row_hbm, col_hbm, tr_hbm, tc_hbm, tpre_hbm, relp_hbm,
                 ridx_v, cidx_v, a0, a1, a2, b0, b1, b2, t0, t1, t2,
                 r0, r1, r2, sg0, sg1, sg2, sw0, sw1, sw2):
    c = lax.axis_index("c")
    s = lax.axis_index("s")
    base0 = c * EPC + s * EPT
    abufs = (a0, a1, a2)
    bbufs = (b0, b1, b2)
    tbufs = (t0, t1, t2)
    rbufs = (r0, r1, r2)
    gsems = (sg0, sg1, sg2)
    wsems = (sw0, sw1, sw2)

    pltpu.sync_copy(row_hbm.at[pl.ds(base0, EPT)], ridx_v)
    pltpu.sync_copy(col_hbm.at[pl.ds(base0, EPT)], cidx_v)

    def start_gather(cix, bsel):
        ri = ridx_v.at[pl.ds(cix * GCHUNK, GCHUNK)]
        ci = cidx_v.at[pl.ds(cix * GCHUNK, GCHUNK)]
        pltpu.async_copy(tr_hbm.at[ri], abufs[bsel], gsems[bsel])
        pltpu.async_copy(tc_hbm.at[ci], bbufs[bsel], gsems[bsel])

    def wait_gather(cix, bsel):
        ri = ridx_v.at[pl.ds(cix * GCHUNK, GCHUNK)]
        pltpu.make_async_copy(tr_hbm.at[ri], abufs[bsel],
                              gsems[bsel]).wait()
        pltpu.make_async_copy(tr_hbm.at[ri], bbufs[bsel],
                              gsems[bsel]).wait()

    def start_write(cix, bsel):
        base = base0 + cix * GCHUNK
        pltpu.async_copy(tbufs[bsel], tpre_hbm.at[pl.ds(base, GCHUNK), :],
                         wsems[bsel])
        pltpu.async_copy(rbufs[bsel], relp_hbm.at[pl.ds(base, GCHUNK), :],
                         wsems[bsel])

    def wait_write(bsel):
        pltpu.make_async_copy(tbufs[bsel],
                              tpre_hbm.at[pl.ds(base0, GCHUNK), :],
                              wsems[bsel]).wait()
        pltpu.make_async_copy(rbufs[bsel],
                              relp_hbm.at[pl.ds(base0, GCHUNK), :],
                              wsems[bsel]).wait()

    def compute(bsel):
        a_v, b_v, t_v, r_v = abufs[bsel], bbufs[bsel], tbufs[bsel], rbufs[bsel]

        def rowop(i, carry2):
            for l in range(D // 16):
                sl = pl.ds(l * 16, 16)
                t_v[i, sl] = a_v[i, sl] + b_v[i, sl]
            sl = pl.ds(D, 16)
            r_v[i, :] = a_v[i, sl] - b_v[i, sl]
            return carry2

        lax.fori_loop(0, GCHUNK, rowop, 0)

    def step(cix, bsel):
        wait_gather(cix, bsel)

        @pl.when(cix >= 3)
        def _():
            wait_write(bsel)

        compute(bsel)
        start_write(cix, bsel)

        @pl.when(cix + 3 < GNCH)
        def _():
            start_gather(cix + 3, bsel)

    start_gather(0, 0)
    start_gather(1, 1)
    start_gather(2, 2)

    def outer(k, carry):
        step(3 * k, 0)
        step(3 * k + 1, 1)
        step(3 * k + 2, 2)
        return carry

    lax.fori_loop(0, GNCH // 3, outer, 0)
    step(GNCH - 1, 0)
    wait_write(0)
    wait_write(1)
    wait_write(2)


def _sc_gather(row, col, t_r, t_c):
    return pl.kernel(
        _gather_body,
        out_type=[
            jax.ShapeDtypeStruct((E, D), jnp.float32),
            jax.ShapeDtypeStruct((E, CPAD), jnp.float32),
        ],
        mesh=plsc.VectorSubcoreMesh(core_axis_name="c", subcore_axis_name="s",
                                    num_cores=SC_CORES,
                                    num_subcores=SC_TILES),
        scratch_types=[
            pltpu.VMEM((EPT,), jnp.int32),
            pltpu.VMEM((EPT,), jnp.int32),
            pltpu.VMEM((GCHUNK, TW), jnp.float32),
            pltpu.VMEM((GCHUNK, TW), jnp.float32),
            pltpu.VMEM((GCHUNK, TW), jnp.float32),
            pltpu.VMEM((GCHUNK, TW), jnp.float32),
            pltpu.VMEM((GCHUNK, TW), jnp.float32),
            pltpu.VMEM((GCHUNK, TW), jnp.float32),
            pltpu.VMEM((GCHUNK, D), jnp.float32),
            pltpu.VMEM((GCHUNK, D), jnp.float32),
            pltpu.VMEM((GCHUNK, D), jnp.float32),
            pltpu.VMEM((GCHUNK, CPAD), jnp.float32),
            pltpu.VMEM((GCHUNK, CPAD), jnp.float32),
            pltpu.VMEM((GCHUNK, CPAD), jnp.float32),
            pltpu.SemaphoreType.DMA,
            pltpu.SemaphoreType.DMA,
            pltpu.SemaphoreType.DMA,
            pltpu.SemaphoreType.DMA,
            pltpu.SemaphoreType.DMA,
            pltpu.SemaphoreType.DMA,
        ],
    )(row, col, t_r, t_c)


def _scatter_body(idx3_hbm, m2_hbm, tr16_hbm, zm_hbm, outm_hbm, outc_hbm,
                  accm, idx_v, m0, m1, t0, sl0, sl1):
    c = lax.axis_index("c")
    s = lax.axis_index("s")
    w = c * SC_TILES + s
    base0 = c * EPC + s * EPT
    mbufs = (m0, m1)
    lsems = (sl0, sl1)

    pltpu.sync_copy(idx3_hbm.at[w], idx_v)

    pltpu.sync_copy(zm_hbm.at[pl.ds(s * NPT, NPT), :],
                    accm.at[pl.ds(s * NPT, NPT), :])
    plsc.subcore_barrier()

    def start_load_m(j, bsel):
        base = base0 + j * CHUNK
        pltpu.async_copy(m2_hbm.at[pl.ds(base, CHUNK), :], mbufs[bsel],
                         lsems[bsel])

    def step_m(j, bsel):
        pltpu.make_async_copy(m2_hbm.at[pl.ds(base0, CHUNK), :],
                              mbufs[bsel], lsems[bsel]).wait()
        pltpu.sync_copy(mbufs[bsel], accm.at[idx_v.at[j]], add=True)

        @pl.when(j + 2 < NCH)
        def _():
            start_load_m(j + 2, bsel)

    start_load_m(0, 0)
    start_load_m(1, 1)

    def loop_m(k, carry):
        step_m(2 * k, 0)
        step_m(2 * k + 1, 1)
        return carry

    lax.fori_loop(0, NCH // 2, loop_m, 0)
    step_m(NCH - 1, 0)
    plsc.subcore_barrier()
    pltpu.sync_copy(accm.at[pl.ds(s * NPT, NPT), :],
                    outm_hbm.at[pl.ds(c * NP + s * NPT, NPT), :])
    plsc.subcore_barrier()

    pltpu.sync_copy(zm_hbm.at[pl.ds(s * NPT, NPT), :],
                    accm.at[pl.ds(s * NPT, NPT), :])
    pltpu.sync_copy(zm_hbm.at[pl.ds(0, CHUNK), :], m0)
    pltpu.sync_copy(zm_hbm.at[pl.ds(0, CHUNK), :], m1)
    plsc.subcore_barrier()

    def start_load_t(j):
        base = base0 + j * CHUNK
        pltpu.async_copy(tr16_hbm.at[pl.ds(base, CHUNK), :], t0, sl0)

    def step_t(j, bsel):
        pltpu.make_async_copy(tr16_hbm.at[pl.ds(base0, CHUNK), :],
                              t0, sl0).wait()
        tw_v = mbufs[bsel]

        def rowop(i, carry2):
            tw_v[i, pl.ds(0, 16)] = t0[i, :]
            return carry2

        lax.fori_loop(0, CHUNK, rowop, 0)

        @pl.when(j + 1 < NCH)
        def _():
            start_load_t(j + 1)

        pltpu.sync_copy(tw_v, accm.at[idx_v.at[j]], add=True)

    start_load_t(0)

    def loop_t(k, carry):
        step_t(2 * k, 0)
        step_t(2 * k + 1, 1)
        return carry

    lax.fori_loop(0, NCH // 2, loop_t, 0)
    step_t(NCH - 1, 0)
    plsc.subcore_barrier()
    pltpu.sync_copy(accm.at[pl.ds(s * NPT, NPT), :],
                    outc_hbm.at[pl.ds(c * NP + s * NPT, NPT), :])


def _sc_scatter(row, m2, transp, zm):
    idx3 = row.reshape(SC_CORES * SC_TILES, NCH, CHUNK)
    return pl.kernel(
        _scatter_body,
        out_type=[
            jax.ShapeDtypeStruct((SC_CORES * NP, D), jnp.float32),
            jax.ShapeDtypeStruct((SC_CORES * NP, D), jnp.float32),
        ],
        mesh=plsc.VectorSubcoreMesh(core_axis_name="c", subcore_axis_name="s",
                                    num_cores=SC_CORES,
                                    num_subcores=SC_TILES),
        scratch_types=[
            pltpu.VMEM_SHARED((NP, D), jnp.float32),
            pltpu.VMEM((NCH, CHUNK), jnp.int32),
            pltpu.VMEM((CHUNK, D), jnp.float32),
            pltpu.VMEM((CHUNK, D), jnp.float32),
            pltpu.VMEM((CHUNK, CPAD), jnp.float32),
            pltpu.SemaphoreType.DMA,
            pltpu.SemaphoreType.DMA,
        ],
    )(idx3, m2, transp, zm)


def _prep_body(af_ref, coordp_ref, atab_ref, w1a_ref, w1b_ref,
               h_ref, tr_ref, tc_ref):
    ids = af_ref[0, 0, :]
    oh = (lax.broadcast_in_dim(ids, (NODE_BLK, NTYPES), (0,))
          == lax.broadcasted_iota(jnp.int32, (NODE_BLK, NTYPES), 1)
          ).astype(jnp.float32)
    h = jnp.dot(oh, atab_ref[...], preferred_element_type=jnp.float32)
    h_ref[...] = h
    z = jnp.zeros((NODE_BLK, TW - D - CPAD), jnp.float32)
    cp = coordp_ref[...]
    tr_ref[...] = jnp.concatenate(
        [jnp.dot(h, w1a_ref[...], preferred_element_type=jnp.float32),
         cp, z], axis=1)
    tc_ref[...] = jnp.concatenate(
        [jnp.dot(h, w1b_ref[...], preferred_element_type=jnp.float32),
         cp, z], axis=1)


def _edge_body(tpre_ref, relp_ref, et_ref, eap_ref, w1d_ref, w2_ref, b2_ref,
               wc1_ref, bc1_ref, wc2_ref, bc2_ref, etab_ref,
               m2_ref, tr_ref, ea_ref):
    relp = relp_ref[...]
    d2 = jnp.sum(relp * relp, axis=1, keepdims=True)
    ids = et_ref[0, 0, :]
    oh = (lax.broadcast_in_dim(ids, (EDGE_BLK, 8), (0,))
          == lax.broadcasted_iota(jnp.int32, (EDGE_BLK, 8), 1)
          ).astype(jnp.float32)
    t1 = (tpre_ref[...] + d2 * w1d_ref[...]
          + jnp.dot(oh, eap_ref[...], preferred_element_type=jnp.float32))
    m = jax.nn.silu(t1)
    m2 = jax.nn.silu(
        jnp.dot(m.astype(jnp.bfloat16), w2_ref[...],
                preferred_element_type=jnp.float32)
        + b2_ref[...])
    c1 = jax.nn.silu(
        jnp.dot(m2.astype(jnp.bfloat16), wc1_ref[...],
                preferred_element_type=jnp.float32)
        + bc1_ref[...])
    cw = jnp.dot(c1, wc2_ref[...], preferred_element_type=jnp.float32) \
        + bc2_ref[...]
    m2_ref[...] = m2
    tr_ref[...] = relp * cw
    ea_ref[...] = jnp.dot(oh, etab_ref[...],
                          preferred_element_type=jnp.float32)


def _node_body(h_ref, agg0_ref, agg1_ref, wn1a_ref, wn1b_ref, bn1_ref,
               wn2_ref, bn2_ref, out_ref):
    h = h_ref[...]
    agg = agg0_ref[...] + agg1_ref[...]
    t = (jnp.dot(h, wn1a_ref[...], preferred_element_type=jnp.float32)
         + jnp.dot(agg, wn1b_ref[...], preferred_element_type=jnp.float32)
         + bn1_ref[...])
    out_ref[...] = h + jnp.dot(jax.nn.silu(t), wn2_ref[...],
                               preferred_element_type=jnp.float32) \
        + bn2_ref[...]


def _full(shape):
    return pl.BlockSpec(shape, lambda i: tuple(0 for _ in shape))


def kernel(atom_feats, coord, edge_index, edge_type_ids, atom_table,
           edge_table, W1, b1, W2, b2, Wc1, bc1, Wc2, bc2, Wn1, bn1, Wn2,
           bn2):
    row = edge_index[0]
    col = edge_index[1]

    coordp = jnp.pad(coord, ((0, 0), (0, CPAD - 3)))
    af3 = atom_feats.reshape(N // NODE_BLK, 1, NODE_BLK)
    et3 = edge_type_ids.reshape(E // EDGE_BLK, 1, EDGE_BLK)
    eap8 = jnp.pad(edge_table @ W1[2 * D + 1:] + b1, ((0, 8 - 4), (0, 0)))
    etab8 = jnp.pad(edge_table, ((0, 8 - 4), (0, 0)))
    w1d = W1[2 * D][None, :]

    h, t_r, t_c = pl.pallas_call(
        _prep_body,
        grid=(N // NODE_BLK,),
        in_specs=[
            pl.BlockSpec((1, 1, NODE_BLK), lambda i: (i, 0, 0)),
            pl.BlockSpec((NODE_BLK, CPAD), lambda i: (i, 0)),
            _full((NTYPES, D)),
            _full((D, D)),
            _full((D, D)),
        ],
        out_specs=[
            pl.BlockSpec((NODE_BLK, D), lambda i: (i, 0)),
            pl.BlockSpec((NODE_BLK, TW), lambda i: (i, 0)),
            pl.BlockSpec((NODE_BLK, TW), lambda i: (i, 0)),
        ],
        out_shape=[
            jax.ShapeDtypeStruct((N, D), jnp.float32),
            jax.ShapeDtypeStruct((N, TW), jnp.float32),
            jax.ShapeDtypeStruct((N, TW), jnp.float32),
        ],
    )(af3, coordp, atom_table, W1[:D], W1[D:2 * D])

    tpre, relp = _sc_gather(row, col, t_r, t_c)

    m2, transp, edge_attr = pl.pallas_call(
        _edge_body,
        grid=(E // EDGE_BLK,),
        in_specs=[
            pl.BlockSpec((EDGE_BLK, D), lambda i: (i, 0)),
            pl.BlockSpec((EDGE_BLK, CPAD), lambda i: (i, 0)),
            pl.BlockSpec((1, 1, EDGE_BLK), lambda i: (i, 0, 0)),
            _full((8, D)),
            _full((1, D)),
            _full((D, D)),
            _full((D,)),
            _full((D, D)),
            _full((D,)),
            _full((D, 1)),
            _full((1,)),
            _full((8, DE)),
        ],
        out_specs=[
            pl.BlockSpec((EDGE_BLK, D), lambda i: (i, 0)),
            pl.BlockSpec((EDGE_BLK, CPAD), lambda i: (i, 0)),
            pl.BlockSpec((EDGE_BLK, DE), lambda i: (i, 0)),
        ],
        out_shape=[
            jax.ShapeDtypeStruct((E, D), jnp.float32),
            jax.ShapeDtypeStruct((E, CPAD), jnp.float32),
            jax.ShapeDtypeStruct((E, DE), jnp.float32),
        ],
    )(tpre, relp, et3, eap8, w1d, W2.astype(jnp.bfloat16), b2,
      Wc1.astype(jnp.bfloat16), bc1, Wc2, bc2, etab8)

    zm = jnp.zeros((NP, D), jnp.float32)
    outm, outc = _sc_scatter(row, m2, transp, zm)
    aggm = outm.reshape(SC_CORES, NP, D)
    aggc = outc.reshape(SC_CORES, NP, D)
    coord_out = coord + (aggc[0, :N, :3] + aggc[1, :N, :3]) / C_NORM

    h_out = pl.pallas_call(
        _node_body,
        grid=(N // NODE_BLK,),
        in_specs=[
            pl.BlockSpec((NODE_BLK, D), lambda i: (i, 0)),
            pl.BlockSpec((NODE_BLK, D), lambda i: (i, 0)),
            pl.BlockSpec((NODE_BLK, D), lambda i: (i, 0)),
            _full((D, D)),
            _full((D, D)),
            _full((D,)),
            _full((D, D)),
            _full((D,)),
        ],
        out_specs=pl.BlockSpec((NODE_BLK, D), lambda i: (i, 0)),
        out_shape=jax.ShapeDtypeStruct((N, D), jnp.float32),
    )(h, aggm[0, :N], aggm[1, :N], Wn1[:D], Wn1[D:], bn1, Wn2, bn2)

    return (h_out, coord_out, edge_attr)


# speedup vs baseline: 4.8989x; 1.0122x over previous
import jax
import jax.numpy as jnp
from jax import lax
from jax.experimental import pallas as pl
from jax.experimental.pallas import tpu as pltpu
from jax.experimental.pallas import tpu_sc as plsc

N = 10000
E = 320000
D = 128
DE = 16
C_NORM = 32.0
NTYPES = 16

EDGE_BLK = 2000
NODE_BLK = 2000
E2 = E // 2
TW = 2 * D
CPAD = 16
RW = 32

SC_CORES = 2
SC_TILES = 16
CHUNK = 40
EPC = E2 // SC_CORES
EPT = EPC // SC_TILES
NCH = EPT // CHUNK
NP = 10240
NPT = NP // SC_TILES
GCHUNK = 40
GNCH = EPT // GCHUNK
TWH = TW // 2
DH = D // 2
RWH = RW // 2


def _gather_body(row_hbm, col_hbm, tr_hbm, tc_hbm, tpre_hbm, relp_hbm,
                 ridx_v, cidx_v, a0, a1, a2, b0, b1, b2, t0, t1, t2,
                 r0, r1, r2, sg0, sg1, sg2, sw0, sw1, sw2):
    c = lax.axis_index("c")
    s = lax.axis_index("s")
    base0 = c * EPC + s * EPT
    abufs = (a0, a1, a2)
    bbufs = (b0, b1, b2)
    tbufs = (t0, t1, t2)
    rbufs = (r0, r1, r2)
    gsems = (sg0, sg1, sg2)
    wsems = (sw0, sw1, sw2)

    pltpu.sync_copy(row_hbm.at[pl.ds(base0, EPT)], ridx_v)
    pltpu.sync_copy(col_hbm.at[pl.ds(base0, EPT)], cidx_v)

    def start_gather(cix, bsel):
        ri = ridx_v.at[pl.ds(cix * GCHUNK, GCHUNK)]
        ci = cidx_v.at[pl.ds(cix * GCHUNK, GCHUNK)]
        pltpu.async_copy(tr_hbm.at[ri], abufs[bsel], gsems[bsel])
        pltpu.async_copy(tc_hbm.at[ci], bbufs[bsel], gsems[bsel])

    def wait_gather(cix, bsel):
        ri = ridx_v.at[pl.ds(cix * GCHUNK, GCHUNK)]
        pltpu.make_async_copy(tr_hbm.at[ri], abufs[bsel],
                              gsems[bsel]).wait()
        pltpu.make_async_copy(tr_hbm.at[ri], bbufs[bsel],
                              gsems[bsel]).wait()

    def start_write(cix, bsel):
        base = base0 + cix * GCHUNK
        pltpu.async_copy(tbufs[bsel], tpre_hbm.at[pl.ds(base, GCHUNK), :],
                         wsems[bsel])
        pltpu.async_copy(rbufs[bsel], relp_hbm.at[pl.ds(base, GCHUNK), :],
                         wsems[bsel])

    def wait_write(bsel):
        pltpu.make_async_copy(tbufs[bsel],
                              tpre_hbm.at[pl.ds(base0, GCHUNK), :],
                              wsems[bsel]).wait()
        pltpu.make_async_copy(rbufs[bsel],
                              relp_hbm.at[pl.ds(base0, GCHUNK), :],
                              wsems[bsel]).wait()

    def compute(bsel):
        a_v, b_v, t_v, r_v = abufs[bsel], bbufs[bsel], tbufs[bsel], rbufs[bsel]

        def rowop(i, carry2):
            for l in range(D // 16):
                sl = pl.ds(l * 16, 16)
                t_v[i, sl] = a_v[i, sl] + b_v[i, sl]
            sl = pl.ds(D, 16)
            r_v[i, :] = a_v[i, sl] - b_v[i, sl]
            return carry2

        lax.fori_loop(0, GCHUNK, rowop, 0)

    def step(cix, bsel):
        wait_gather(cix, bsel)

        @pl.when(cix >= 3)
        def _():
            wait_write(bsel)

        compute(bsel)
        start_write(cix, bsel)

        @pl.when(cix + 3 < GNCH)
        def _():
            start_gather(cix + 3, bsel)

    start_gather(0, 0)
    start_gather(1, 1)
    start_gather(2, 2)

    def outer(k, carry):
        step(3 * k, 0)
        step(3 * k + 1, 1)
        step(3 * k + 2, 2)
        return carry

    lax.fori_loop(0, GNCH // 3, outer, 0)
    for cix in range(3 * (GNCH // 3), GNCH):
        step(cix, cix % 3)
    wait_write(0)
    wait_write(1)
    wait_write(2)


def _sc_gather(row, col, t_r, t_c):
    return pl.kernel(
        _gather_body,
        out_type=[
            jax.ShapeDtypeStruct((E2, D), jnp.float32),
            jax.ShapeDtypeStruct((E2, CPAD), jnp.float32),
        ],
        mesh=plsc.VectorSubcoreMesh(core_axis_name="c", subcore_axis_name="s",
                                    num_cores=SC_CORES,
                                    num_subcores=SC_TILES),
        scratch_types=[
            pltpu.VMEM((EPT,), jnp.int32),
            pltpu.VMEM((EPT,), jnp.int32),
            pltpu.VMEM((GCHUNK, TW), jnp.float32),
            pltpu.VMEM((GCHUNK, TW), jnp.float32),
            pltpu.VMEM((GCHUNK, TW), jnp.float32),
            pltpu.VMEM((GCHUNK, TW), jnp.float32),
            pltpu.VMEM((GCHUNK, TW), jnp.float32),
            pltpu.VMEM((GCHUNK, TW), jnp.float32),
            pltpu.VMEM((GCHUNK, D), jnp.float32),
            pltpu.VMEM((GCHUNK, D), jnp.float32),
            pltpu.VMEM((GCHUNK, D), jnp.float32),
            pltpu.VMEM((GCHUNK, CPAD), jnp.float32),
            pltpu.VMEM((GCHUNK, CPAD), jnp.float32),
            pltpu.VMEM((GCHUNK, CPAD), jnp.float32),
            pltpu.SemaphoreType.DMA,
            pltpu.SemaphoreType.DMA,
            pltpu.SemaphoreType.DMA,
            pltpu.SemaphoreType.DMA,
            pltpu.SemaphoreType.DMA,
            pltpu.SemaphoreType.DMA,
        ],
    )(row, col, t_r, t_c)


def _scatter_body(idx3_hbm, m2_hbm, tr16_hbm, zm_hbm, outm_hbm, outc_hbm,
                  accm, idx_v, m0, m1, t0, sl0, sl1):
    c = lax.axis_index("c")
    s = lax.axis_index("s")
    w = c * SC_TILES + s
    base0 = c * EPC + s * EPT
    mbufs = (m0, m1)
    lsems = (sl0, sl1)

    pltpu.sync_copy(idx3_hbm.at[w], idx_v)

    pltpu.sync_copy(zm_hbm.at[pl.ds(s * NPT, NPT), :],
                    accm.at[pl.ds(s * NPT, NPT), :])
    plsc.subcore_barrier()

    def start_load_m(j, bsel):
        base = base0 + j * CHUNK
        pltpu.async_copy(m2_hbm.at[pl.ds(base, CHUNK), :], mbufs[bsel],
                         lsems[bsel])

    def step_m(j, bsel):
        pltpu.make_async_copy(m2_hbm.at[pl.ds(base0, CHUNK), :],
                              mbufs[bsel], lsems[bsel]).wait()
        pltpu.sync_copy(mbufs[bsel], accm.at[idx_v.at[j]], add=True)

        @pl.when(j + 2 < NCH)
        def _():
            start_load_m(j + 2, bsel)

    start_load_m(0, 0)
    start_load_m(1, 1)

    def loop_m(k, carry):
        step_m(2 * k, 0)
        step_m(2 * k + 1, 1)
        return carry

    lax.fori_loop(0, NCH // 2, loop_m, 0)
    step_m(NCH - 1, 0)
    plsc.subcore_barrier()
    pltpu.sync_copy(accm.at[pl.ds(s * NPT, NPT), :],
                    outm_hbm.at[pl.ds(c * NP + s * NPT, NPT), :])
    plsc.subcore_barrier()

    pltpu.sync_copy(zm_hbm.at[pl.ds(s * NPT, NPT), :],
                    accm.at[pl.ds(s * NPT, NPT), :])
    pltpu.sync_copy(zm_hbm.at[pl.ds(0, CHUNK), :], m0)
    pltpu.sync_copy(zm_hbm.at[pl.ds(0, CHUNK), :], m1)
    plsc.subcore_barrier()

    def start_load_t(j):
        base = base0 + j * CHUNK
        pltpu.async_copy(tr16_hbm.at[pl.ds(base, CHUNK), :], t0, sl0)

    def step_t(j, bsel):
        pltpu.make_async_copy(tr16_hbm.at[pl.ds(base0, CHUNK), :],
                              t0, sl0).wait()
        tw_v = mbufs[bsel]

        def rowop(i, carry2):
            tw_v[i, pl.ds(0, 16)] = t0[i, :]
            return carry2

        lax.fori_loop(0, CHUNK, rowop, 0)

        @pl.when(j + 1 < NCH)
        def _():
            start_load_t(j + 1)

        pltpu.sync_copy(tw_v, accm.at[idx_v.at[j]], add=True)

    start_load_t(0)

    def loop_t(k, carry):
        step_t(2 * k, 0)
        step_t(2 * k + 1, 1)
        return carry

    lax.fori_loop(0, NCH // 2, loop_t, 0)
    step_t(NCH - 1, 0)
    plsc.subcore_barrier()
    pltpu.sync_copy(accm.at[pl.ds(s * NPT, NPT), :],
                    outc_hbm.at[pl.ds(c * NP + s * NPT, NPT), :])


def _sc_scatter(row, m2, transp, zm):
    idx3 = row.reshape(SC_CORES * SC_TILES, NCH, CHUNK)
    return pl.kernel(
        _scatter_body,
        out_type=[
            jax.ShapeDtypeStruct((SC_CORES * NP, D), jnp.float32),
            jax.ShapeDtypeStruct((SC_CORES * NP, D), jnp.float32),
        ],
        mesh=plsc.VectorSubcoreMesh(core_axis_name="c", subcore_axis_name="s",
                                    num_cores=SC_CORES,
                                    num_subcores=SC_TILES),
        scratch_types=[
            pltpu.VMEM_SHARED((NP, D), jnp.float32),
            pltpu.VMEM((NCH, CHUNK), jnp.int32),
            pltpu.VMEM((CHUNK, D), jnp.float32),
            pltpu.VMEM((CHUNK, D), jnp.float32),
            pltpu.VMEM((CHUNK, CPAD), jnp.float32),
            pltpu.SemaphoreType.DMA,
            pltpu.SemaphoreType.DMA,
        ],
    )(idx3, m2, transp, zm)


def _prep_body(af_ref, coordp_ref, atab_ref, w1a_ref, w1b_ref,
               h_ref, tr_ref, tc_ref):
    ids = af_ref[0, 0, :]
    oh = (lax.broadcast_in_dim(ids, (NODE_BLK, NTYPES), (0,))
          == lax.broadcasted_iota(jnp.int32, (NODE_BLK, NTYPES), 1)
          ).astype(jnp.float32)
    h = jnp.dot(oh, atab_ref[...], preferred_element_type=jnp.float32)
    h_ref[...] = h
    z = jnp.zeros((NODE_BLK, TW - D - CPAD), jnp.float32)
    cp = coordp_ref[...]
    tr_ref[...] = jnp.concatenate(
        [jnp.dot(h, w1a_ref[...], preferred_element_type=jnp.float32),
         cp, z], axis=1)
    tc_ref[...] = jnp.concatenate(
        [jnp.dot(h, w1b_ref[...], preferred_element_type=jnp.float32),
         cp, z], axis=1)


def _edge_body(tpre_ref, relp_ref, et_ref, eap_ref, w1d_ref, w2_ref, b2_ref,
               wc1_ref, bc1_ref, wc2_ref, bc2_ref, etab_ref,
               m2_ref, tr_ref, ea_ref):
    relp = relp_ref[...]
    d2 = jnp.sum(relp * relp, axis=1, keepdims=True)
    ids = et_ref[0, 0, :]
    oh = (lax.broadcast_in_dim(ids, (EDGE_BLK, 8), (0,))
          == lax.broadcasted_iota(jnp.int32, (EDGE_BLK, 8), 1)
          ).astype(jnp.float32)
    t1 = (tpre_ref[...] + d2 * w1d_ref[...]
          + jnp.dot(oh, eap_ref[...], preferred_element_type=jnp.float32))
    m = jax.nn.silu(t1)
    m2 = jax.nn.silu(
        jnp.dot(m.astype(jnp.bfloat16), w2_ref[...],
                preferred_element_type=jnp.float32)
        + b2_ref[...])
    c1 = jax.nn.silu(
        jnp.dot(m2.astype(jnp.bfloat16), wc1_ref[...],
                preferred_element_type=jnp.float32)
        + bc1_ref[...])
    cw = jnp.dot(c1, wc2_ref[...], preferred_element_type=jnp.float32) \
        + bc2_ref[...]
    m2_ref[...] = m2
    tr_ref[...] = relp * cw
    ea_ref[...] = jnp.dot(oh, etab_ref[...],
                          preferred_element_type=jnp.float32)


def _node_body(h_ref, agg0_ref, agg1_ref, agg2_ref, agg3_ref,
               wn1a_ref, wn1b_ref, bn1_ref, wn2_ref, bn2_ref, out_ref):
    h = h_ref[...]
    agg = (agg0_ref[...] + agg1_ref[...] + agg2_ref[...] + agg3_ref[...])
    t = (jnp.dot(h, wn1a_ref[...], preferred_element_type=jnp.float32)
         + jnp.dot(agg, wn1b_ref[...], preferred_element_type=jnp.float32)
         + bn1_ref[...])
    out_ref[...] = h + jnp.dot(jax.nn.silu(t), wn2_ref[...],
                               preferred_element_type=jnp.float32) \
        + bn2_ref[...]


def _full(shape):
    return pl.BlockSpec(shape, lambda i: tuple(0 for _ in shape))


def kernel(atom_feats, coord, edge_index, edge_type_ids, atom_table,
           edge_table, W1, b1, W2, b2, Wc1, bc1, Wc2, bc2, Wn1, bn1, Wn2,
           bn2):
    row = edge_index[0]
    col = edge_index[1]

    coordp = jnp.pad(coord, ((0, 0), (0, CPAD - 3)))
    af3 = atom_feats.reshape(N // NODE_BLK, 1, NODE_BLK)
    eap8 = jnp.pad(edge_table @ W1[2 * D + 1:] + b1, ((0, 8 - 4), (0, 0)))
    etab8 = jnp.pad(edge_table, ((0, 8 - 4), (0, 0)))
    w1d = W1[2 * D][None, :]

    h, t_r, t_c = pl.pallas_call(
        _prep_body,
        grid=(N // NODE_BLK,),
        in_specs=[
            pl.BlockSpec((1, 1, NODE_BLK), lambda i: (i, 0, 0)),
            pl.BlockSpec((NODE_BLK, CPAD), lambda i: (i, 0)),
            _full((NTYPES, D)),
            _full((D, D)),
            _full((D, D)),
        ],
        out_specs=[
            pl.BlockSpec((NODE_BLK, D), lambda i: (i, 0)),
            pl.BlockSpec((NODE_BLK, TW), lambda i: (i, 0)),
            pl.BlockSpec((NODE_BLK, TW), lambda i: (i, 0)),
        ],
        out_shape=[
            jax.ShapeDtypeStruct((N, D), jnp.float32),
            jax.ShapeDtypeStruct((N, TW), jnp.float32),
            jax.ShapeDtypeStruct((N, TW), jnp.float32),
        ],
    )(af3, coordp, atom_table, W1[:D], W1[D:2 * D])

    def edge_call(tpre_h, relp_h, et3_h):
        return pl.pallas_call(
            _edge_body,
            grid=(E2 // EDGE_BLK,),
            in_specs=[
                pl.BlockSpec((EDGE_BLK, D), lambda i: (i, 0)),
                pl.BlockSpec((EDGE_BLK, CPAD), lambda i: (i, 0)),
                pl.BlockSpec((1, 1, EDGE_BLK), lambda i: (i, 0, 0)),
                _full((8, D)),
                _full((1, D)),
                _full((D, D)),
                _full((D,)),
                _full((D, D)),
                _full((D,)),
                _full((D, 1)),
                _full((1,)),
                _full((8, DE)),
            ],
            out_specs=[
                pl.BlockSpec((EDGE_BLK, D), lambda i: (i, 0)),
                pl.BlockSpec((EDGE_BLK, CPAD), lambda i: (i, 0)),
                pl.BlockSpec((EDGE_BLK, DE), lambda i: (i, 0)),
            ],
            out_shape=[
                jax.ShapeDtypeStruct((E2, D), jnp.float32),
                jax.ShapeDtypeStruct((E2, CPAD), jnp.float32),
                jax.ShapeDtypeStruct((E2, DE), jnp.float32),
            ],
        )(tpre_h, relp_h, et3_h, eap8, w1d, W2.astype(jnp.bfloat16), b2,
          Wc1.astype(jnp.bfloat16), bc1, Wc2, bc2, etab8)

    zm = jnp.zeros((NP, D), jnp.float32)
    halves = []
    for hi in range(2):
        sl = slice(hi * E2, (hi + 1) * E2)
        tpre_h, relp_h = _sc_gather(row[sl], col[sl], t_r, t_c)
        et3_h = edge_type_ids[sl].reshape(E2 // EDGE_BLK, 1, EDGE_BLK)
        m2_h, tr_h, ea_h = edge_call(tpre_h, relp_h, et3_h)
        outm_h, outc_h = _sc_scatter(row[sl], m2_h, tr_h, zm)
        halves.append((outm_h.reshape(SC_CORES, NP, D),
                       outc_h.reshape(SC_CORES, NP, D), ea_h))

    (aggm0, aggc0, ea0), (aggm1, aggc1, ea1) = halves
    edge_attr = jnp.concatenate([ea0, ea1], axis=0)
    coord_out = coord + (aggc0[0, :N, :3] + aggc0[1, :N, :3]
                         + aggc1[0, :N, :3] + aggc1[1, :N, :3]) / C_NORM

    h_out = pl.pallas_call(
        _node_body,
        grid=(N // NODE_BLK,),
        in_specs=[
            pl.BlockSpec((NODE_BLK, D), lambda i: (i, 0)),
            pl.BlockSpec((NODE_BLK, D), lambda i: (i, 0)),
            pl.BlockSpec((NODE_BLK, D), lambda i: (i, 0)),
            pl.BlockSpec((NODE_BLK, D), lambda i: (i, 0)),
            pl.BlockSpec((NODE_BLK, D), lambda i: (i, 0)),
            _full((D, D)),
            _full((D, D)),
            _full((D,)),
            _full((D, D)),
            _full((D,)),
        ],
        out_specs=pl.BlockSpec((NODE_BLK, D), lambda i: (i, 0)),
        out_shape=jax.ShapeDtypeStruct((N, D), jnp.float32),
    )(h, aggm0[0, :N], aggm0[1, :N], aggm1[0, :N], aggm1[1, :N],
      Wn1[:D], Wn1[D:], bn1, Wn2, bn2)

    return (h_out, coord_out, edge_attr)
